# Initial kernel scaffold; baseline (speedup 1.0000x reference)
import jax
import jax.numpy as jnp
from jax.experimental import pallas as pl


def kernel(x, edge_index, ec_W1, ec_b1, ec_g1, ec_be1, ec_W2, ec_b2, ec_g2, ec_be2, ec_W3, ec_b3, ec_g3, ec_be3, nd_W1, nd_b1, nd_g1, nd_be1, nd_W2, nd_b2, ed_W1, ed_b1, ed_g1, ed_be1, ed_W2, ed_b2):
    raise NotImplementedError("write your pallas kernel here")


# speedup vs baseline: 2.9581x; 2.9581x over previous
import jax
import jax.numpy as jnp
from jax import lax
from jax.experimental import pallas as pl
from jax.experimental.pallas import tpu as pltpu
from jax.experimental.pallas import tpu_sc as plsc

NC = 2
NS = 16
NW = NC * NS
L = 16
EPB = 128
NPAD = 10240

_F32 = jnp.float32


def _wid():
    return lax.axis_index("s") * NC + lax.axis_index("c")


def _nblk(wid, nb):
    base, rem = nb // NW, nb % NW
    return base + jnp.where(wid < rem, 1, 0).astype(jnp.int32)


def _sc_mesh():
    return plsc.VectorSubcoreMesh(core_axis_name="c", subcore_axis_name="s")


def _k1_body(t_hbm, src_hbm, dst_hbm, z1_hbm, st_hbm,
             idxs_v, idxd_v, bufd_v, bufs_v, bufz_v, acc_v, sem1, sem2):
    wid = _wid()
    nb = 2 * z1_hbm.shape[0] // EPB

    def pair_body(p, carry):
        out = list(carry)
        for half in range(2):
            e = 2 * p + half
            for g in range(4):
                a = bufd_v[e, pl.ds(g * L, L)]
                b = bufs_v[e, pl.ds(64 + g * L, L)]
                z = a + b
                bufz_v[p, pl.ds(half * 64 + g * L, L)] = z
                out[g] = out[g] + z
                out[4 + g] = out[4 + g] + z * z
        return tuple(out)

    def chunk(i, carry):
        blk = i * NW + wid
        eb = blk * EPB
        pltpu.sync_copy(dst_hbm.at[pl.ds(eb, EPB)], idxd_v)
        pltpu.sync_copy(src_hbm.at[pl.ds(eb, EPB)], idxs_v)
        cpa = pltpu.async_copy(t_hbm.at[idxd_v], bufd_v, sem1)
        cpb = pltpu.async_copy(t_hbm.at[idxs_v], bufs_v, sem2)
        cpa.wait()
        cpb.wait()
        carry = lax.fori_loop(0, EPB // 2, pair_body, carry)
        pltpu.sync_copy(bufz_v, z1_hbm.at[pl.ds(blk * (EPB // 2), EPB // 2)])
        return carry

    zero = jnp.zeros((L,), _F32)
    carry = lax.fori_loop(0, _nblk(wid, nb), chunk, (zero,) * 8)
    for g in range(4):
        acc_v[pl.ds(g * L, L)] = carry[g]
        acc_v[pl.ds(64 + g * L, L)] = carry[4 + g]
    pltpu.sync_copy(acc_v, st_hbm.at[pl.ds(wid * 128, 128)])


def _k1(t, src, dst):
    e = src.shape[0]
    return pl.kernel(
        _k1_body,
        out_type=[jax.ShapeDtypeStruct((e // 2, 128), _F32),
                  jax.ShapeDtypeStruct((NW * 128,), _F32)],
        mesh=_sc_mesh(),
        scratch_types=[
            pltpu.VMEM((EPB,), jnp.int32),
            pltpu.VMEM((EPB,), jnp.int32),
            pltpu.VMEM((EPB, 128), _F32),
            pltpu.VMEM((EPB, 128), _F32),
            pltpu.VMEM((EPB // 2, 128), _F32),
            pltpu.VMEM((128,), _F32),
            pltpu.SemaphoreType.DMA,
            pltpu.SemaphoreType.DMA,
        ],
    )(t, src, dst)


def _k6_body(c_hbm, src_hbm, dst_hbm, ze_hbm, st_hbm,
             idxs_v, idxd_v, bufs_v, bufd_v, acc_v, sem1, sem2):
    wid = _wid()
    nb = ze_hbm.shape[0] // EPB

    def edge_body(e, carry):
        out = list(carry)
        for g in range(8):
            s = bufs_v[e, pl.ds(g * L, L)]
            d = bufd_v[e, pl.ds(g * L, L)]
            z = s - d
            bufs_v[e, pl.ds(g * L, L)] = z
            out[g] = out[g] + z
            out[8 + g] = out[8 + g] + z * z
        return tuple(out)

    def chunk(i, carry):
        eb = (i * NW + wid) * EPB
        pltpu.sync_copy(src_hbm.at[pl.ds(eb, EPB)], idxs_v)
        pltpu.sync_copy(dst_hbm.at[pl.ds(eb, EPB)], idxd_v)
        cps = pltpu.async_copy(c_hbm.at[idxs_v], bufs_v, sem1)
        cpd = pltpu.async_copy(c_hbm.at[idxd_v], bufd_v, sem2)
        cps.wait()
        cpd.wait()
        carry = lax.fori_loop(0, EPB, edge_body, carry)
        pltpu.sync_copy(bufs_v, ze_hbm.at[pl.ds(eb, EPB)])
        return carry

    zero = jnp.zeros((L,), _F32)
    carry = lax.fori_loop(0, _nblk(wid, nb), chunk, (zero,) * 16)
    for g in range(8):
        acc_v[pl.ds(g * L, L)] = carry[g]
        acc_v[pl.ds(128 + g * L, L)] = carry[8 + g]
    pltpu.sync_copy(acc_v, st_hbm.at[pl.ds(wid * 256, 256)])


def _k6(c, src, dst):
    e = src.shape[0]
    return pl.kernel(
        _k6_body,
        out_type=[jax.ShapeDtypeStruct((e, 128), _F32),
                  jax.ShapeDtypeStruct((NW * 256,), _F32)],
        mesh=_sc_mesh(),
        scratch_types=[
            pltpu.VMEM((EPB,), jnp.int32),
            pltpu.VMEM((EPB,), jnp.int32),
            pltpu.VMEM((EPB, 128), _F32),
            pltpu.VMEM((EPB, 128), _F32),
            pltpu.VMEM((256,), _F32),
            pltpu.SemaphoreType.DMA,
            pltpu.SemaphoreType.DMA,
        ],
    )(c, src, dst)


def _k4_body(z3_hbm, dst_hbm, scsh_hbm, out_hbm,
             idx_v, bufz_v, bufh_v, scsh_v, s_sh, sem0):
    del sem0
    cid = lax.axis_index("c")
    sid = lax.axis_index("s")
    wid = sid * NC + cid
    nb = 2 * z3_hbm.shape[0] // EPB
    rows_per_sub = NPAD // NS

    pltpu.sync_copy(scsh_hbm, scsh_v)

    zero = jnp.zeros((L,), _F32)

    def zb(i, c):
        for g in range(8):
            bufh_v[i, pl.ds(g * L, L)] = zero
        return c

    lax.fori_loop(0, EPB, zb, 0)
    rb = sid * rows_per_sub
    for r in range(rows_per_sub // EPB):
        pltpu.sync_copy(bufh_v, s_sh.at[pl.ds(rb + r * EPB, EPB)])

    onev = jnp.where(lax.iota(jnp.int32, L) == 0,
                     jnp.full((L,), 1.0, _F32), zero)

    def ob(i, c):
        bufh_v[i, pl.ds(64, L)] = onev
        return c

    lax.fori_loop(0, EPB, ob, 0)
    plsc.subcore_barrier()

    scv = [scsh_v[pl.ds(c * L, L)] for c in range(8)]
    shv = [scsh_v[pl.ds(128 + c * L, L)] for c in range(8)]

    def pair_body(p, c):
        for half in range(2):
            e = 2 * p + half
            for g in range(4):
                z = bufz_v[p, pl.ds(half * 64 + g * L, L)]
                h = jnp.maximum(z * scv[4 * half + g] + shv[4 * half + g],
                                0.0)
                bufh_v[e, pl.ds(g * L, L)] = h
        return c

    def chunk(i, c):
        blk = i * NW + wid
        pltpu.sync_copy(dst_hbm.at[pl.ds(blk * EPB, EPB)], idx_v)
        pltpu.sync_copy(z3_hbm.at[pl.ds(blk * (EPB // 2), EPB // 2)], bufz_v)
        lax.fori_loop(0, EPB // 2, pair_body, 0)
        pltpu.sync_copy(bufh_v, s_sh.at[idx_v], add=True)
        return c

    lax.fori_loop(0, _nblk(wid, nb), chunk, 0)
    plsc.subcore_barrier()

    ob2 = cid * NPAD + rb
    for r in range(rows_per_sub // EPB):
        pltpu.sync_copy(s_sh.at[pl.ds(rb + r * EPB, EPB)],
                        out_hbm.at[pl.ds(ob2 + r * EPB, EPB)])


def _k4(z3p, dst, scsh):
    return pl.kernel(
        _k4_body,
        out_type=jax.ShapeDtypeStruct((NC * NPAD, 128), _F32),
        mesh=_sc_mesh(),
        scratch_types=[
            pltpu.VMEM((EPB,), jnp.int32),
            pltpu.VMEM((EPB // 2, 128), _F32),
            pltpu.VMEM((EPB, 128), _F32),
            pltpu.VMEM((256,), _F32),
            pltpu.VMEM_SHARED((NPAD, 128), _F32),
            pltpu.SemaphoreType.DMA,
        ],
    )(z3p, dst, scsh)


def _k0_body(x_ref, w1_ref, t_ref):
    nf = x_ref.shape[1]
    x = x_ref[...]
    w1a = w1_ref[:nf]
    w1b = w1_ref[nf:]
    t_ref[:, :64] = jnp.dot(x, w1a - w1b, preferred_element_type=_F32)
    t_ref[:, 64:] = jnp.dot(x, w1b, preferred_element_type=_F32)


def _k0(x, w1):
    n = x.shape[0]
    return pl.pallas_call(
        _k0_body,
        out_shape=jax.ShapeDtypeStruct((n, 128), _F32),
    )(x, w1)


def _mlp_body(z_ref, w_ref, scsh_ref, zo_ref, st_ref, acc_ref):
    i = pl.program_id(0)
    h = jnp.maximum(z_ref[...] * scsh_ref[0:1, :] + scsh_ref[1:2, :], 0.0)
    z2 = jnp.dot(h, w_ref[...], preferred_element_type=_F32)
    zo_ref[...] = z2

    @pl.when(i == 0)
    def _():
        acc_ref[...] = jnp.zeros_like(acc_ref)

    acc_ref[0:1, :] += jnp.sum(z2, axis=0, keepdims=True)
    acc_ref[1:2, :] += jnp.sum(z2 * z2, axis=0, keepdims=True)

    @pl.when(i == pl.num_programs(0) - 1)
    def _():
        st_ref[...] = acc_ref[...]


def _mlp_layer(zp, wd, scshp, bp=2000):
    e2 = zp.shape[0]
    grid = e2 // bp
    return pl.pallas_call(
        _mlp_body,
        grid=(grid,),
        in_specs=[
            pl.BlockSpec((bp, 128), lambda i: (i, 0)),
            pl.BlockSpec((128, 128), lambda i: (0, 0)),
            pl.BlockSpec((2, 128), lambda i: (0, 0)),
        ],
        out_specs=[
            pl.BlockSpec((bp, 128), lambda i: (i, 0)),
            pl.BlockSpec((2, 128), lambda i: (0, 0)),
        ],
        out_shape=[jax.ShapeDtypeStruct((e2, 128), _F32),
                   jax.ShapeDtypeStruct((2, 128), _F32)],
        scratch_shapes=[pltpu.VMEM((2, 128), _F32)],
    )(zp, wd, scshp)


def _k5_body(sagg_ref, x_ref, ndw1_ref, ndg_ref, ndbe_ref, ndw2_ref,
             ndb2_ref, edw1_ref, nout_ref, c_ref):
    n = x_ref.shape[0]
    h = ndw1_ref.shape[0] - x_ref.shape[1]
    s = sagg_ref[0:n, 0:h] + sagg_ref[NPAD:NPAD + n, 0:h]
    cnt = sagg_ref[0:n, h:h + 1] + sagg_ref[NPAD:NPAD + n, h:h + 1]
    agg = s / jnp.maximum(cnt, 1.0)
    x = x_ref[...]
    zn = (jnp.dot(agg, ndw1_ref[:h], preferred_element_type=_F32)
          + jnp.dot(x, ndw1_ref[h:], preferred_element_type=_F32))
    m = jnp.mean(zn, axis=0, keepdims=True)
    v = jnp.mean(zn * zn, axis=0, keepdims=True) - m * m
    hn = jnp.maximum((zn - m) * lax.rsqrt(v + 1e-5) * ndg_ref[...]
                     + ndbe_ref[...], 0.0)
    logit = jnp.dot(hn, ndw2_ref[...], preferred_element_type=_F32) \
        + ndb2_ref[...]
    nout_ref[...] = 1.0 / (1.0 + jnp.exp(-logit))
    c_ref[...] = (jnp.dot(agg, edw1_ref[:h], preferred_element_type=_F32)
                  + jnp.dot(x, edw1_ref[h:], preferred_element_type=_F32))


def _k5(sagg, x, ndw1, ndg, ndbe, ndw2, ndb2, edw1):
    n = x.shape[0]
    mid = ndw1.shape[1]
    return pl.pallas_call(
        _k5_body,
        out_shape=[jax.ShapeDtypeStruct((n, 1), _F32),
                   jax.ShapeDtypeStruct((n, mid), _F32)],
    )(sagg, x, ndw1, ndg, ndbe, ndw2, ndb2, edw1)


def _k7_body(ze_ref, scsh_ref, w2_ref, b2_ref, out_ref):
    he = jnp.maximum(ze_ref[...] * scsh_ref[0:1, :] + scsh_ref[1:2, :], 0.0)
    logit = jnp.dot(he, w2_ref[...], preferred_element_type=_F32) \
        + b2_ref[...]
    out_ref[...] = 1.0 / (1.0 + jnp.exp(-logit))


def _k7(ze, scsh, w2, b2, be=4000):
    e, h = ze.shape
    grid = e // be
    return pl.pallas_call(
        _k7_body,
        grid=(grid,),
        in_specs=[
            pl.BlockSpec((be, h), lambda i: (i, 0)),
            pl.BlockSpec((2, h), lambda i: (0, 0)),
            pl.BlockSpec((h, 1), lambda i: (0, 0)),
            pl.BlockSpec((1, 1), lambda i: (0, 0)),
        ],
        out_specs=pl.BlockSpec((be, 1), lambda i: (i, 0)),
        out_shape=jax.ShapeDtypeStruct((e, 1), _F32),
    )(ze, scsh, w2, b2)


def _scale_shift(sums, sumsq, e, g, be):
    m = sums / e
    v = sumsq / e - m * m
    sc = g * lax.rsqrt(v + 1e-5)
    return sc, be - m * sc


def _blockdiag(w):
    h = w.shape[0]
    z = jnp.zeros((h, h), _F32)
    return jnp.block([[w, z], [z, w]])


def kernel(x, edge_index, ec_W1, ec_b1, ec_g1, ec_be1, ec_W2, ec_b2, ec_g2,
           ec_be2, ec_W3, ec_b3, ec_g3, ec_be3, nd_W1, nd_b1, nd_g1, nd_be1,
           nd_W2, nd_b2, ed_W1, ed_b1, ed_g1, ed_be1, ed_W2, ed_b2):
    del ec_b1, ec_b2, ec_b3, nd_b1, ed_b1
    n = x.shape[0]
    e = edge_index.shape[1]
    ef = jnp.float32(e)
    src = edge_index[0]
    dst = edge_index[1]

    t = _k0(x, ec_W1)
    z1p, st1 = _k1(t, src, dst)

    st1 = st1.reshape(NW, 128).sum(axis=0)
    sc1, sh1 = _scale_shift(st1[:64], st1[64:], ef, ec_g1, ec_be1)
    scsh1p = jnp.stack([jnp.tile(sc1, 2), jnp.tile(sh1, 2)])
    z2p, st2 = _mlp_layer(z1p, _blockdiag(ec_W2), scsh1p)

    sc2, sh2 = _scale_shift(st2[0, :64] + st2[0, 64:],
                            st2[1, :64] + st2[1, 64:], ef, ec_g2, ec_be2)
    scsh2p = jnp.stack([jnp.tile(sc2, 2), jnp.tile(sh2, 2)])
    z3p, st3 = _mlp_layer(z2p, _blockdiag(ec_W3), scsh2p)

    sc3, sh3 = _scale_shift(st3[0, :64] + st3[0, 64:],
                            st3[1, :64] + st3[1, 64:], ef, ec_g3, ec_be3)
    scsh3p = jnp.concatenate([jnp.tile(sc3, 2), jnp.tile(sh3, 2)])
    sagg = _k4(z3p, dst, scsh3p)

    node_out, c = _k5(sagg, x, nd_W1, nd_g1.reshape(1, -1),
                      nd_be1.reshape(1, -1), nd_W2, nd_b2.reshape(1, 1),
                      ed_W1)

    ze, ste = _k6(c, src, dst)
    ste = ste.reshape(NW, 256).sum(axis=0)
    sce, she = _scale_shift(ste[:128], ste[128:], ef, ed_g1, ed_be1)
    edge_out = _k7(ze, jnp.stack([sce, she]), ed_W2, ed_b2.reshape(1, 1))
    return (node_out, edge_out)


# speedup vs baseline: 3.7825x; 1.2787x over previous
import jax
import jax.numpy as jnp
from jax import lax
from jax.experimental import pallas as pl
from jax.experimental.pallas import tpu as pltpu
from jax.experimental.pallas import tpu_sc as plsc

NC = 2
NS = 16
NW = NC * NS
L = 16
EPB = 128
B4 = 64
NPAD = 10240

_F32 = jnp.float32


def _wid():
    return lax.axis_index("s") * NC + lax.axis_index("c")


def _sc_mesh():
    return plsc.VectorSubcoreMesh(core_axis_name="c", subcore_axis_name="s")


def _drain(src, dst, sem):
    pltpu.make_async_copy(src, dst, sem).wait()


def _k1_body(t_hbm, src_hbm, dst_hbm, z1_hbm, st_hbm,
             idxd0, idxd1, idxs0, idxs1, bufd0, bufd1, bufs0, bufs1,
             bufz0, bufz1, idxtd, idxts, buftd, bufts, bufzt, acc_v,
             gd0, gd1, gs0, gs1, w0, w1, ts0, ts1):
    wid = _wid()
    e = 2 * z1_hbm.shape[0]
    epw = e // NW
    nblk = epw // EPB
    tail = epw - nblk * EPB
    wbase = wid * epw
    zbase = wid * (epw // 2)
    idxd, idxs = [idxd0, idxd1], [idxs0, idxs1]
    bufd, bufs = [bufd0, bufd1], [bufs0, bufs1]
    bufz = [bufz0, bufz1]
    gsemd, gsems, wsem = [gd0, gd1], [gs0, gs1], [w0, w1]

    def make_pair_body(bd, bs, bz):
        def pair_body(p, carry):
            out = list(carry)
            for half in range(2):
                ei = 2 * p + half
                for g in range(4):
                    a = bd[ei, pl.ds(g * L, L)]
                    b = bs[ei, pl.ds(64 + g * L, L)]
                    z = a + b
                    bz[p, pl.ds(half * 64 + g * L, L)] = z
                    out[g] = out[g] + z
                    out[4 + g] = out[4 + g] + z * z
            return tuple(out)
        return pair_body

    def fire(j, sj):
        eb = wbase + j * EPB
        pltpu.sync_copy(dst_hbm.at[pl.ds(eb, EPB)], idxd[sj])
        pltpu.sync_copy(src_hbm.at[pl.ds(eb, EPB)], idxs[sj])
        pltpu.async_copy(t_hbm.at[idxd[sj]], bufd[sj], gsemd[sj])
        pltpu.async_copy(t_hbm.at[idxs[sj]], bufs[sj], gsems[sj])

    fire(0, 0)

    def halfiter(i, s, carry):
        @pl.when(i < nblk - 1)
        def _():
            fire(i + 1, 1 - s)

        _drain(t_hbm.at[pl.ds(0, EPB)], bufd[s], gsemd[s])
        _drain(t_hbm.at[pl.ds(0, EPB)], bufs[s], gsems[s])

        @pl.when(i >= 2)
        def _():
            _drain(bufz[s], z1_hbm.at[pl.ds(0, EPB // 2)], wsem[s])

        carry = lax.fori_loop(0, EPB // 2,
                              make_pair_body(bufd[s], bufs[s], bufz[s]),
                              carry)
        pltpu.async_copy(
            bufz[s], z1_hbm.at[pl.ds(zbase + i * (EPB // 2), EPB // 2)],
            wsem[s])
        return carry

    def outer(k, carry):
        carry = halfiter(2 * k, 0, carry)
        carry = halfiter(2 * k + 1, 1, carry)
        return carry

    zero = jnp.zeros((L,), _F32)
    carry = lax.fori_loop(0, nblk // 2, outer, (zero,) * 8)
    _drain(bufz[0], z1_hbm.at[pl.ds(0, EPB // 2)], wsem[0])
    _drain(bufz[1], z1_hbm.at[pl.ds(0, EPB // 2)], wsem[1])

    tb = wbase + nblk * EPB
    pltpu.sync_copy(dst_hbm.at[pl.ds(tb, tail)], idxtd)
    pltpu.sync_copy(src_hbm.at[pl.ds(tb, tail)], idxts)
    cpa = pltpu.async_copy(t_hbm.at[idxtd], buftd, ts0)
    cpb = pltpu.async_copy(t_hbm.at[idxts], bufts, ts1)
    cpa.wait()
    cpb.wait()
    carry = lax.fori_loop(0, tail // 2,
                          make_pair_body(buftd, bufts, bufzt), carry)
    pltpu.sync_copy(bufzt,
                    z1_hbm.at[pl.ds(zbase + nblk * (EPB // 2), tail // 2)])

    for g in range(4):
        acc_v[pl.ds(g * L, L)] = carry[g]
        acc_v[pl.ds(64 + g * L, L)] = carry[4 + g]
    pltpu.sync_copy(acc_v, st_hbm.at[pl.ds(wid * 128, 128)])


def _k1(t, src, dst):
    e = src.shape[0]
    tail = (e // NW) % EPB
    return pl.kernel(
        _k1_body,
        out_type=[jax.ShapeDtypeStruct((e // 2, 128), _F32),
                  jax.ShapeDtypeStruct((NW * 128,), _F32)],
        mesh=_sc_mesh(),
        scratch_types=[
            pltpu.VMEM((EPB,), jnp.int32),
            pltpu.VMEM((EPB,), jnp.int32),
            pltpu.VMEM((EPB,), jnp.int32),
            pltpu.VMEM((EPB,), jnp.int32),
            pltpu.VMEM((EPB, 128), _F32),
            pltpu.VMEM((EPB, 128), _F32),
            pltpu.VMEM((EPB, 128), _F32),
            pltpu.VMEM((EPB, 128), _F32),
            pltpu.VMEM((EPB // 2, 128), _F32),
            pltpu.VMEM((EPB // 2, 128), _F32),
            pltpu.VMEM((tail,), jnp.int32),
            pltpu.VMEM((tail,), jnp.int32),
            pltpu.VMEM((tail, 128), _F32),
            pltpu.VMEM((tail, 128), _F32),
            pltpu.VMEM((tail // 2, 128), _F32),
            pltpu.VMEM((128,), _F32),
            pltpu.SemaphoreType.DMA,
            pltpu.SemaphoreType.DMA,
            pltpu.SemaphoreType.DMA,
            pltpu.SemaphoreType.DMA,
            pltpu.SemaphoreType.DMA,
            pltpu.SemaphoreType.DMA,
            pltpu.SemaphoreType.DMA,
            pltpu.SemaphoreType.DMA,
        ],
    )(t, src, dst)


def _k6_body(c_hbm, src_hbm, dst_hbm, ze_hbm, st_hbm,
             idxd0, idxd1, idxs0, idxs1, bufd0, bufd1, bufs0, bufs1,
             bufz0, bufz1, idxtd, idxts, buftd, bufts, bufzt, acc_v,
             gd0, gd1, gs0, gs1, w0, w1, ts0, ts1):
    wid = _wid()
    e = ze_hbm.shape[0]
    epw = e // NW
    nblk = epw // EPB
    tail = epw - nblk * EPB
    wbase = wid * epw
    idxd, idxs = [idxd0, idxd1], [idxs0, idxs1]
    bufd, bufs = [bufd0, bufd1], [bufs0, bufs1]
    bufz = [bufz0, bufz1]
    gsemd, gsems, wsem = [gd0, gd1], [gs0, gs1], [w0, w1]

    def make_edge_body(bd, bs, bz):
        def edge_body(ei, carry):
            out = list(carry)
            for g in range(8):
                sv = bs[ei, pl.ds(g * L, L)]
                dv = bd[ei, pl.ds(g * L, L)]
                z = sv - dv
                bz[ei, pl.ds(g * L, L)] = z
                out[g] = out[g] + z
                out[8 + g] = out[8 + g] + z * z
            return tuple(out)
        return edge_body

    def fire(j, sj):
        eb = wbase + j * EPB
        pltpu.sync_copy(dst_hbm.at[pl.ds(eb, EPB)], idxd[sj])
        pltpu.sync_copy(src_hbm.at[pl.ds(eb, EPB)], idxs[sj])
        pltpu.async_copy(c_hbm.at[idxd[sj]], bufd[sj], gsemd[sj])
        pltpu.async_copy(c_hbm.at[idxs[sj]], bufs[sj], gsems[sj])

    fire(0, 0)

    def halfiter(i, s, carry):
        @pl.when(i < nblk - 1)
        def _():
            fire(i + 1, 1 - s)

        _drain(c_hbm.at[pl.ds(0, EPB)], bufd[s], gsemd[s])
        _drain(c_hbm.at[pl.ds(0, EPB)], bufs[s], gsems[s])

        @pl.when(i >= 2)
        def _():
            _drain(bufz[s], ze_hbm.at[pl.ds(0, EPB)], wsem[s])

        carry = lax.fori_loop(0, EPB,
                              make_edge_body(bufd[s], bufs[s], bufz[s]),
                              carry)
        pltpu.async_copy(bufz[s], ze_hbm.at[pl.ds(wbase + i * EPB, EPB)],
                         wsem[s])
        return carry

    def outer(k, carry):
        carry = halfiter(2 * k, 0, carry)
        carry = halfiter(2 * k + 1, 1, carry)
        return carry

    zero = jnp.zeros((L,), _F32)
    carry = lax.fori_loop(0, nblk // 2, outer, (zero,) * 16)
    _drain(bufz[0], ze_hbm.at[pl.ds(0, EPB)], wsem[0])
    _drain(bufz[1], ze_hbm.at[pl.ds(0, EPB)], wsem[1])

    tb = wbase + nblk * EPB
    pltpu.sync_copy(dst_hbm.at[pl.ds(tb, tail)], idxtd)
    pltpu.sync_copy(src_hbm.at[pl.ds(tb, tail)], idxts)
    cpa = pltpu.async_copy(c_hbm.at[idxtd], buftd, ts0)
    cpb = pltpu.async_copy(c_hbm.at[idxts], bufts, ts1)
    cpa.wait()
    cpb.wait()
    carry = lax.fori_loop(0, tail,
                          make_edge_body(buftd, bufts, bufzt), carry)
    pltpu.sync_copy(bufzt, ze_hbm.at[pl.ds(tb, tail)])

    for g in range(8):
        acc_v[pl.ds(g * L, L)] = carry[g]
        acc_v[pl.ds(128 + g * L, L)] = carry[8 + g]
    pltpu.sync_copy(acc_v, st_hbm.at[pl.ds(wid * 256, 256)])


def _k6(c, src, dst):
    e = src.shape[0]
    tail = (e // NW) % EPB
    return pl.kernel(
        _k6_body,
        out_type=[jax.ShapeDtypeStruct((e, 128), _F32),
                  jax.ShapeDtypeStruct((NW * 256,), _F32)],
        mesh=_sc_mesh(),
        scratch_types=[
            pltpu.VMEM((EPB,), jnp.int32),
            pltpu.VMEM((EPB,), jnp.int32),
            pltpu.VMEM((EPB,), jnp.int32),
            pltpu.VMEM((EPB,), jnp.int32),
            pltpu.VMEM((EPB, 128), _F32),
            pltpu.VMEM((EPB, 128), _F32),
            pltpu.VMEM((EPB, 128), _F32),
            pltpu.VMEM((EPB, 128), _F32),
            pltpu.VMEM((EPB, 128), _F32),
            pltpu.VMEM((EPB, 128), _F32),
            pltpu.VMEM((tail,), jnp.int32),
            pltpu.VMEM((tail,), jnp.int32),
            pltpu.VMEM((tail, 128), _F32),
            pltpu.VMEM((tail, 128), _F32),
            pltpu.VMEM((tail, 128), _F32),
            pltpu.VMEM((256,), _F32),
            pltpu.SemaphoreType.DMA,
            pltpu.SemaphoreType.DMA,
            pltpu.SemaphoreType.DMA,
            pltpu.SemaphoreType.DMA,
            pltpu.SemaphoreType.DMA,
            pltpu.SemaphoreType.DMA,
            pltpu.SemaphoreType.DMA,
            pltpu.SemaphoreType.DMA,
        ],
    )(c, src, dst)


def _k4_body(z3_hbm, dst_hbm, scsh_hbm, out_hbm,
             idx0, idx1, bufz0, bufz1, bufh0, bufh1,
             idxt, bufzt, bufht, scsh_v, s_sh,
             r0, r1, sc0, sc1, ts0):
    cid = lax.axis_index("c")
    sid = lax.axis_index("s")
    wid = sid * NC + cid
    B = B4
    e = 2 * z3_hbm.shape[0]
    epw = e // NW
    nblk = epw // B
    tail = epw - nblk * B
    wbase = wid * epw
    zbase = wid * (epw // 2)
    rows_per_sub = NPAD // NS
    idx = [idx0, idx1]
    bufz = [bufz0, bufz1]
    bufh = [bufh0, bufh1]
    rsem, ssem = [r0, r1], [sc0, sc1]

    pltpu.sync_copy(scsh_hbm, scsh_v)

    zero = jnp.zeros((L,), _F32)

    def zb(i, c):
        for g in range(8):
            bufh0[i, pl.ds(g * L, L)] = zero
            bufh1[i, pl.ds(g * L, L)] = zero
        return c

    lax.fori_loop(0, B, zb, 0)
    rb = sid * rows_per_sub
    for r in range(rows_per_sub // B):
        pltpu.sync_copy(bufh0, s_sh.at[pl.ds(rb + r * B, B)])

    onev = jnp.where(lax.iota(jnp.int32, L) == 0,
                     jnp.full((L,), 1.0, _F32), zero)

    def ob(i, c):
        bufh0[i, pl.ds(64, L)] = onev
        bufh1[i, pl.ds(64, L)] = onev
        return c

    lax.fori_loop(0, B, ob, 0)

    def obt(i, c):
        for g in range(8):
            bufht[i, pl.ds(g * L, L)] = zero
        return c

    lax.fori_loop(0, tail, obt, 0)

    def obt2(i, c):
        bufht[i, pl.ds(64, L)] = onev
        return c

    lax.fori_loop(0, tail, obt2, 0)
    plsc.subcore_barrier()

    scv = [scsh_v[pl.ds(c * L, L)] for c in range(8)]
    shv = [scsh_v[pl.ds(128 + c * L, L)] for c in range(8)]

    def make_pair_body(bz, bh):
        def pair_body(p, c):
            for half in range(2):
                ei = 2 * p + half
                for g in range(4):
                    z = bz[p, pl.ds(half * 64 + g * L, L)]
                    h = jnp.maximum(
                        z * scv[4 * half + g] + shv[4 * half + g], 0.0)
                    bh[ei, pl.ds(g * L, L)] = h
            return c
        return pair_body

    def fire(j, sj):
        @pl.when(j >= 2)
        def _():
            _drain(bufh[sj], s_sh.at[pl.ds(0, B)], ssem[sj])

        pltpu.sync_copy(dst_hbm.at[pl.ds(wbase + j * B, B)], idx[sj])
        pltpu.async_copy(
            z3_hbm.at[pl.ds(zbase + j * (B // 2), B // 2)],
            bufz[sj], rsem[sj])

    fire(0, 0)

    def halfiter(i, s, c):
        @pl.when(i < nblk - 1)
        def _():
            fire(i + 1, 1 - s)

        _drain(z3_hbm.at[pl.ds(0, B // 2)], bufz[s], rsem[s])
        lax.fori_loop(0, B // 2, make_pair_body(bufz[s], bufh[s]), 0)
        pltpu.async_copy(bufh[s], s_sh.at[idx[s]], ssem[s], add=True)
        return c

    def outer(k, c):
        c = halfiter(2 * k, 0, c)
        c = halfiter(2 * k + 1, 1, c)
        return c

    lax.fori_loop(0, nblk // 2, outer, 0)
    _drain(bufh[0], s_sh.at[pl.ds(0, B)], ssem[0])
    _drain(bufh[1], s_sh.at[pl.ds(0, B)], ssem[1])

    tb = wbase + nblk * B
    pltpu.sync_copy(dst_hbm.at[pl.ds(tb, tail)], idxt)
    pltpu.sync_copy(z3_hbm.at[pl.ds(zbase + nblk * (B // 2), tail // 2)],
                    bufzt)
    lax.fori_loop(0, tail // 2, make_pair_body(bufzt, bufht), 0)
    pltpu.sync_copy(bufht, s_sh.at[idxt], add=True)

    plsc.subcore_barrier()

    ob2 = cid * NPAD + rb
    for r in range(rows_per_sub // B):
        pltpu.sync_copy(s_sh.at[pl.ds(rb + r * B, B)],
                        out_hbm.at[pl.ds(ob2 + r * B, B)])


def _k4(z3p, dst, scsh):
    e = dst.shape[0]
    tail = (e // NW) % B4
    return pl.kernel(
        _k4_body,
        out_type=jax.ShapeDtypeStruct((NC * NPAD, 128), _F32),
        mesh=_sc_mesh(),
        scratch_types=[
            pltpu.VMEM((B4,), jnp.int32),
            pltpu.VMEM((B4,), jnp.int32),
            pltpu.VMEM((B4 // 2, 128), _F32),
            pltpu.VMEM((B4 // 2, 128), _F32),
            pltpu.VMEM((B4, 128), _F32),
            pltpu.VMEM((B4, 128), _F32),
            pltpu.VMEM((tail,), jnp.int32),
            pltpu.VMEM((tail // 2, 128), _F32),
            pltpu.VMEM((tail, 128), _F32),
            pltpu.VMEM((256,), _F32),
            pltpu.VMEM_SHARED((NPAD, 128), _F32),
            pltpu.SemaphoreType.DMA,
            pltpu.SemaphoreType.DMA,
            pltpu.SemaphoreType.DMA,
            pltpu.SemaphoreType.DMA,
            pltpu.SemaphoreType.DMA,
        ],
    )(z3p, dst, scsh)


def _k0_body(x_ref, w1_ref, t_ref):
    nf = x_ref.shape[1]
    x = x_ref[...]
    w1a = w1_ref[:nf]
    w1b = w1_ref[nf:]
    t_ref[:, :64] = jnp.dot(x, w1a - w1b, preferred_element_type=_F32)
    t_ref[:, 64:] = jnp.dot(x, w1b, preferred_element_type=_F32)


def _k0(x, w1):
    n = x.shape[0]
    return pl.pallas_call(
        _k0_body,
        out_shape=jax.ShapeDtypeStruct((n, 128), _F32),
    )(x, w1)


def _mlp_body(z_ref, w_ref, scsh_ref, zo_ref, st_ref, acc_ref):
    i = pl.program_id(0)
    h = jnp.maximum(z_ref[...] * scsh_ref[0:1, :] + scsh_ref[1:2, :], 0.0)
    z2 = jnp.dot(h, w_ref[...], preferred_element_type=_F32)
    zo_ref[...] = z2

    @pl.when(i == 0)
    def _():
        acc_ref[...] = jnp.zeros_like(acc_ref)

    acc_ref[0:1, :] += jnp.sum(z2, axis=0, keepdims=True)
    acc_ref[1:2, :] += jnp.sum(z2 * z2, axis=0, keepdims=True)

    @pl.when(i == pl.num_programs(0) - 1)
    def _():
        st_ref[...] = acc_ref[...]


def _mlp_layer(zp, wd, scshp, bp=2000):
    e2 = zp.shape[0]
    grid = e2 // bp
    return pl.pallas_call(
        _mlp_body,
        grid=(grid,),
        in_specs=[
            pl.BlockSpec((bp, 128), lambda i: (i, 0)),
            pl.BlockSpec((128, 128), lambda i: (0, 0)),
            pl.BlockSpec((2, 128), lambda i: (0, 0)),
        ],
        out_specs=[
            pl.BlockSpec((bp, 128), lambda i: (i, 0)),
            pl.BlockSpec((2, 128), lambda i: (0, 0)),
        ],
        out_shape=[jax.ShapeDtypeStruct((e2, 128), _F32),
                   jax.ShapeDtypeStruct((2, 128), _F32)],
        scratch_shapes=[pltpu.VMEM((2, 128), _F32)],
    )(zp, wd, scshp)


def _k5_body(sagg_ref, x_ref, ndw1_ref, ndg_ref, ndbe_ref, ndw2_ref,
             ndb2_ref, edw1_ref, nout_ref, c_ref):
    n = x_ref.shape[0]
    h = ndw1_ref.shape[0] - x_ref.shape[1]
    s = sagg_ref[0:n, 0:h] + sagg_ref[NPAD:NPAD + n, 0:h]
    cnt = sagg_ref[0:n, h:h + 1] + sagg_ref[NPAD:NPAD + n, h:h + 1]
    agg = s / jnp.maximum(cnt, 1.0)
    x = x_ref[...]
    zn = (jnp.dot(agg, ndw1_ref[:h], preferred_element_type=_F32)
          + jnp.dot(x, ndw1_ref[h:], preferred_element_type=_F32))
    m = jnp.mean(zn, axis=0, keepdims=True)
    v = jnp.mean(zn * zn, axis=0, keepdims=True) - m * m
    hn = jnp.maximum((zn - m) * lax.rsqrt(v + 1e-5) * ndg_ref[...]
                     + ndbe_ref[...], 0.0)
    logit = jnp.dot(hn, ndw2_ref[...], preferred_element_type=_F32) \
        + ndb2_ref[...]
    nout_ref[...] = 1.0 / (1.0 + jnp.exp(-logit))
    c_ref[...] = (jnp.dot(agg, edw1_ref[:h], preferred_element_type=_F32)
                  + jnp.dot(x, edw1_ref[h:], preferred_element_type=_F32))


def _k5(sagg, x, ndw1, ndg, ndbe, ndw2, ndb2, edw1):
    n = x.shape[0]
    mid = ndw1.shape[1]
    return pl.pallas_call(
        _k5_body,
        out_shape=[jax.ShapeDtypeStruct((n, 1), _F32),
                   jax.ShapeDtypeStruct((n, mid), _F32)],
    )(sagg, x, ndw1, ndg, ndbe, ndw2, ndb2, edw1)


def _k7_body(ze_ref, scsh_ref, w2_ref, b2_ref, out_ref):
    he = jnp.maximum(ze_ref[...] * scsh_ref[0:1, :] + scsh_ref[1:2, :], 0.0)
    logit = jnp.dot(he, w2_ref[...], preferred_element_type=_F32) \
        + b2_ref[...]
    out_ref[...] = 1.0 / (1.0 + jnp.exp(-logit))


def _k7(ze, scsh, w2, b2, be=4000):
    e, h = ze.shape
    grid = e // be
    return pl.pallas_call(
        _k7_body,
        grid=(grid,),
        in_specs=[
            pl.BlockSpec((be, h), lambda i: (i, 0)),
            pl.BlockSpec((2, h), lambda i: (0, 0)),
            pl.BlockSpec((h, 1), lambda i: (0, 0)),
            pl.BlockSpec((1, 1), lambda i: (0, 0)),
        ],
        out_specs=pl.BlockSpec((be, 1), lambda i: (i, 0)),
        out_shape=jax.ShapeDtypeStruct((e, 1), _F32),
    )(ze, scsh, w2, b2)


def _scale_shift(sums, sumsq, e, g, be):
    m = sums / e
    v = sumsq / e - m * m
    sc = g * lax.rsqrt(v + 1e-5)
    return sc, be - m * sc


def _blockdiag(w):
    h = w.shape[0]
    z = jnp.zeros((h, h), _F32)
    return jnp.block([[w, z], [z, w]])


def kernel(x, edge_index, ec_W1, ec_b1, ec_g1, ec_be1, ec_W2, ec_b2, ec_g2,
           ec_be2, ec_W3, ec_b3, ec_g3, ec_be3, nd_W1, nd_b1, nd_g1, nd_be1,
           nd_W2, nd_b2, ed_W1, ed_b1, ed_g1, ed_be1, ed_W2, ed_b2):
    del ec_b1, ec_b2, ec_b3, nd_b1, ed_b1
    e = edge_index.shape[1]
    ef = jnp.float32(e)
    src = edge_index[0]
    dst = edge_index[1]

    t = _k0(x, ec_W1)
    z1p, st1 = _k1(t, src, dst)

    st1 = st1.reshape(NW, 128).sum(axis=0)
    sc1, sh1 = _scale_shift(st1[:64], st1[64:], ef, ec_g1, ec_be1)
    scsh1p = jnp.stack([jnp.tile(sc1, 2), jnp.tile(sh1, 2)])
    z2p, st2 = _mlp_layer(z1p, _blockdiag(ec_W2), scsh1p)

    sc2, sh2 = _scale_shift(st2[0, :64] + st2[0, 64:],
                            st2[1, :64] + st2[1, 64:], ef, ec_g2, ec_be2)
    scsh2p = jnp.stack([jnp.tile(sc2, 2), jnp.tile(sh2, 2)])
    z3p, st3 = _mlp_layer(z2p, _blockdiag(ec_W3), scsh2p)

    sc3, sh3 = _scale_shift(st3[0, :64] + st3[0, 64:],
                            st3[1, :64] + st3[1, 64:], ef, ec_g3, ec_be3)
    scsh3p = jnp.concatenate([jnp.tile(sc3, 2), jnp.tile(sh3, 2)])
    sagg = _k4(z3p, dst, scsh3p)

    node_out, c = _k5(sagg, x, nd_W1, nd_g1.reshape(1, -1),
                      nd_be1.reshape(1, -1), nd_W2, nd_b2.reshape(1, 1),
                      ed_W1)

    ze, ste = _k6(c, src, dst)
    ste = ste.reshape(NW, 256).sum(axis=0)
    sce, she = _scale_shift(ste[:128], ste[128:], ef, ed_g1, ed_be1)
    edge_out = _k7(ze, jnp.stack([sce, she]), ed_W2, ed_b2.reshape(1, 1))
    return (node_out, edge_out)


# speedup vs baseline: 4.0157x; 1.0616x over previous
import jax
import jax.numpy as jnp
from jax import lax
from jax.experimental import pallas as pl
from jax.experimental.pallas import tpu as pltpu
from jax.experimental.pallas import tpu_sc as plsc

NC = 2
NS = 16
NW = NC * NS
L = 16
EPB = 128
B4 = 64
NPAD = 10240

_F32 = jnp.float32


def _wid():
    return lax.axis_index("s") * NC + lax.axis_index("c")


def _sc_mesh():
    return plsc.VectorSubcoreMesh(core_axis_name="c", subcore_axis_name="s")


def _drain(src, dst, sem):
    pltpu.make_async_copy(src, dst, sem).wait()


def _k1_body(t_hbm, ei_hbm, z1_hbm, st_hbm,
             idx0, idx1, bufd0, bufd1, bufs0, bufs1,
             bufz0, bufz1, acc_v,
             gd0, gd1, gs0, gs1, w0, w1):
    wid = _wid()
    e = 2 * z1_hbm.shape[0]
    nb_total = e // EPB
    nmain = nb_total // NW
    nxw = nb_total % NW
    epw = nmain * EPB
    wbase = wid * epw
    zbase = wid * (epw // 2)
    has_x = wid < nxw
    xeb = jnp.where(has_x, NW * epw + wid * EPB, wbase)
    xzoff = jnp.where(has_x, NW * (epw // 2) + wid * (EPB // 2), zbase)
    flagf = jnp.where(has_x, jnp.full((L,), 1.0, _F32),
                      jnp.zeros((L,), _F32))
    idx = [idx0, idx1]
    bufd, bufs = [bufd0, bufd1], [bufs0, bufs1]
    bufz = [bufz0, bufz1]
    gsemd, gsems, wsem = [gd0, gd1], [gs0, gs1], [w0, w1]

    def make_pair_body(bd, bs, bz, scale):
        def pair_body(p, carry):
            out = list(carry)
            for half in range(2):
                row = 2 * p + half
                for g in range(4):
                    a = bd[row, pl.ds(g * L, L)]
                    b = bs[row, pl.ds(64 + g * L, L)]
                    z = a + b
                    bz[p, pl.ds(half * 64 + g * L, L)] = z
                    zs = z if scale is None else z * scale
                    out[g] = out[g] + zs
                    out[4 + g] = out[4 + g] + z * zs
            return tuple(out)
        return pair_body

    def fire(j, sj):
        eb = jnp.where(j == nmain, xeb, wbase + j * EPB)
        pltpu.sync_copy(ei_hbm.at[:, pl.ds(eb, EPB)], idx[sj])
        pltpu.async_copy(t_hbm.at[idx[sj].at[1]], bufd[sj], gsemd[sj])
        pltpu.async_copy(t_hbm.at[idx[sj].at[0]], bufs[sj], gsems[sj])

    fire(0, 0)

    def halfiter(i, s, carry):
        @pl.when(i < nmain)
        def _():
            fire(i + 1, 1 - s)

        _drain(t_hbm.at[pl.ds(0, EPB)], bufd[s], gsemd[s])
        _drain(t_hbm.at[pl.ds(0, EPB)], bufs[s], gsems[s])

        @pl.when(i >= 2)
        def _():
            _drain(bufz[s], z1_hbm.at[pl.ds(0, EPB // 2)], wsem[s])

        carry = lax.fori_loop(
            0, EPB // 2, make_pair_body(bufd[s], bufs[s], bufz[s], None),
            carry)
        pltpu.async_copy(
            bufz[s], z1_hbm.at[pl.ds(zbase + i * (EPB // 2), EPB // 2)],
            wsem[s])
        return carry

    def outer(k, carry):
        carry = halfiter(2 * k, 0, carry)
        carry = halfiter(2 * k + 1, 1, carry)
        return carry

    zero = jnp.zeros((L,), _F32)
    carry = lax.fori_loop(0, nmain // 2, outer, (zero,) * 8)

    _drain(t_hbm.at[pl.ds(0, EPB)], bufd[0], gsemd[0])
    _drain(t_hbm.at[pl.ds(0, EPB)], bufs[0], gsems[0])
    _drain(bufz[0], z1_hbm.at[pl.ds(0, EPB // 2)], wsem[0])
    carry = lax.fori_loop(
        0, EPB // 2, make_pair_body(bufd[0], bufs[0], bufz[0], flagf), carry)
    pltpu.async_copy(bufz[0], z1_hbm.at[pl.ds(xzoff, EPB // 2)], wsem[0])

    _drain(bufz[0], z1_hbm.at[pl.ds(0, EPB // 2)], wsem[0])
    _drain(bufz[1], z1_hbm.at[pl.ds(0, EPB // 2)], wsem[1])

    for g in range(4):
        acc_v[pl.ds(g * L, L)] = carry[g]
        acc_v[pl.ds(64 + g * L, L)] = carry[4 + g]
    pltpu.sync_copy(acc_v, st_hbm.at[pl.ds(wid * 128, 128)])


def _k1(t, ei):
    e = ei.shape[1]
    return pl.kernel(
        _k1_body,
        out_type=[jax.ShapeDtypeStruct((e // 2, 128), _F32),
                  jax.ShapeDtypeStruct((NW * 128,), _F32)],
        mesh=_sc_mesh(),
        scratch_types=[
            pltpu.VMEM((2, EPB), jnp.int32),
            pltpu.VMEM((2, EPB), jnp.int32),
            pltpu.VMEM((EPB, 128), _F32),
            pltpu.VMEM((EPB, 128), _F32),
            pltpu.VMEM((EPB, 128), _F32),
            pltpu.VMEM((EPB, 128), _F32),
            pltpu.VMEM((EPB // 2, 128), _F32),
            pltpu.VMEM((EPB // 2, 128), _F32),
            pltpu.VMEM((128,), _F32),
            pltpu.SemaphoreType.DMA,
            pltpu.SemaphoreType.DMA,
            pltpu.SemaphoreType.DMA,
            pltpu.SemaphoreType.DMA,
            pltpu.SemaphoreType.DMA,
            pltpu.SemaphoreType.DMA,
        ],
    )(t, ei)


def _k6_body(c_hbm, ei_hbm, ze_hbm, st_hbm,
             idx0, idx1, bufd0, bufd1, bufs0, bufs1,
             bufz0, bufz1, acc_v,
             gd0, gd1, gs0, gs1, w0, w1):
    wid = _wid()
    e = ze_hbm.shape[0]
    nb_total = e // EPB
    nmain = nb_total // NW
    nxw = nb_total % NW
    epw = nmain * EPB
    wbase = wid * epw
    has_x = wid < nxw
    xeb = jnp.where(has_x, NW * epw + wid * EPB, wbase)
    flagf = jnp.where(has_x, jnp.full((L,), 1.0, _F32),
                      jnp.zeros((L,), _F32))
    idx = [idx0, idx1]
    bufd, bufs = [bufd0, bufd1], [bufs0, bufs1]
    bufz = [bufz0, bufz1]
    gsemd, gsems, wsem = [gd0, gd1], [gs0, gs1], [w0, w1]

    def make_edge_body(bd, bs, bz, scale):
        def edge_body(row, carry):
            out = list(carry)
            for g in range(8):
                sv = bs[row, pl.ds(g * L, L)]
                dv = bd[row, pl.ds(g * L, L)]
                z = sv - dv
                bz[row, pl.ds(g * L, L)] = z
                zs = z if scale is None else z * scale
                out[g] = out[g] + zs
                out[8 + g] = out[8 + g] + z * zs
            return tuple(out)
        return edge_body

    def fire(j, sj):
        eb = jnp.where(j == nmain, xeb, wbase + j * EPB)
        pltpu.sync_copy(ei_hbm.at[:, pl.ds(eb, EPB)], idx[sj])
        pltpu.async_copy(c_hbm.at[idx[sj].at[1]], bufd[sj], gsemd[sj])
        pltpu.async_copy(c_hbm.at[idx[sj].at[0]], bufs[sj], gsems[sj])

    fire(0, 0)

    def halfiter(i, s, carry):
        @pl.when(i < nmain)
        def _():
            fire(i + 1, 1 - s)

        _drain(c_hbm.at[pl.ds(0, EPB)], bufd[s], gsemd[s])
        _drain(c_hbm.at[pl.ds(0, EPB)], bufs[s], gsems[s])

        @pl.when(i >= 2)
        def _():
            _drain(bufz[s], ze_hbm.at[pl.ds(0, EPB)], wsem[s])

        carry = lax.fori_loop(
            0, EPB, make_edge_body(bufd[s], bufs[s], bufz[s], None), carry)
        pltpu.async_copy(bufz[s], ze_hbm.at[pl.ds(wbase + i * EPB, EPB)],
                         wsem[s])
        return carry

    def outer(k, carry):
        carry = halfiter(2 * k, 0, carry)
        carry = halfiter(2 * k + 1, 1, carry)
        return carry

    zero = jnp.zeros((L,), _F32)
    carry = lax.fori_loop(0, nmain // 2, outer, (zero,) * 16)

    _drain(c_hbm.at[pl.ds(0, EPB)], bufd[0], gsemd[0])
    _drain(c_hbm.at[pl.ds(0, EPB)], bufs[0], gsems[0])
    _drain(bufz[0], ze_hbm.at[pl.ds(0, EPB)], wsem[0])
    carry = lax.fori_loop(
        0, EPB, make_edge_body(bufd[0], bufs[0], bufz[0], flagf), carry)
    pltpu.async_copy(bufz[0], ze_hbm.at[pl.ds(xeb, EPB)], wsem[0])

    _drain(bufz[0], ze_hbm.at[pl.ds(0, EPB)], wsem[0])
    _drain(bufz[1], ze_hbm.at[pl.ds(0, EPB)], wsem[1])

    for g in range(8):
        acc_v[pl.ds(g * L, L)] = carry[g]
        acc_v[pl.ds(128 + g * L, L)] = carry[8 + g]
    pltpu.sync_copy(acc_v, st_hbm.at[pl.ds(wid * 256, 256)])


def _k6(c, ei):
    e = ei.shape[1]
    return pl.kernel(
        _k6_body,
        out_type=[jax.ShapeDtypeStruct((e, 128), _F32),
                  jax.ShapeDtypeStruct((NW * 256,), _F32)],
        mesh=_sc_mesh(),
        scratch_types=[
            pltpu.VMEM((2, EPB), jnp.int32),
            pltpu.VMEM((2, EPB), jnp.int32),
            pltpu.VMEM((EPB, 128), _F32),
            pltpu.VMEM((EPB, 128), _F32),
            pltpu.VMEM((EPB, 128), _F32),
            pltpu.VMEM((EPB, 128), _F32),
            pltpu.VMEM((EPB, 128), _F32),
            pltpu.VMEM((EPB, 128), _F32),
            pltpu.VMEM((256,), _F32),
            pltpu.SemaphoreType.DMA,
            pltpu.SemaphoreType.DMA,
            pltpu.SemaphoreType.DMA,
            pltpu.SemaphoreType.DMA,
            pltpu.SemaphoreType.DMA,
            pltpu.SemaphoreType.DMA,
        ],
    )(c, ei)


def _k4_body(z3_hbm, dst_hbm, scsh_hbm, out_hbm,
             idx0, idx1, bufz0, bufz1, bufh0, bufh1, scsh_v, s_sh,
             r0, r1, sc0, sc1):
    cid = lax.axis_index("c")
    sid = lax.axis_index("s")
    wid = sid * NC + cid
    B = B4
    e = 2 * z3_hbm.shape[0]
    nb_total = e // B
    nmain = nb_total // NW
    nxw = nb_total % NW
    epw = nmain * B
    wbase = wid * epw
    has_x = wid < nxw
    xeb = jnp.where(has_x, NW * epw + wid * B, wbase)
    flagf = jnp.where(has_x, jnp.full((L,), 1.0, _F32),
                      jnp.zeros((L,), _F32))
    rows_per_sub = NPAD // NS
    idx = [idx0, idx1]
    bufz = [bufz0, bufz1]
    bufh = [bufh0, bufh1]
    rsem, ssem = [r0, r1], [sc0, sc1]

    pltpu.sync_copy(scsh_hbm, scsh_v)

    zero = jnp.zeros((L,), _F32)

    def zb(i, c):
        for g in range(5):
            bufh0[i, pl.ds(g * L, L)] = zero
            bufh1[i, pl.ds(g * L, L)] = zero
        return c

    lax.fori_loop(0, B, zb, 0)
    rb = sid * rows_per_sub
    for r in range(rows_per_sub // B):
        pltpu.sync_copy(bufh0, s_sh.at[pl.ds(rb + r * B, B)])

    onev = jnp.where(lax.iota(jnp.int32, L) == 0,
                     jnp.full((L,), 1.0, _F32), zero)

    def ob(i, c):
        bufh0[i, pl.ds(64, L)] = onev
        bufh1[i, pl.ds(64, L)] = onev
        return c

    lax.fori_loop(0, B, ob, 0)
    plsc.subcore_barrier()

    scv = [scsh_v[pl.ds(c * L, L)] for c in range(8)]
    shv = [scsh_v[pl.ds(128 + c * L, L)] for c in range(8)]

    def make_pair_body(bz, bh, scale):
        def pair_body(p, c):
            for half in range(2):
                row = 2 * p + half
                for g in range(4):
                    z = bz[p, pl.ds(half * 64 + g * L, L)]
                    h = jnp.maximum(
                        z * scv[4 * half + g] + shv[4 * half + g], 0.0)
                    if scale is not None:
                        h = h * scale
                    bh[row, pl.ds(g * L, L)] = h
            return c
        return pair_body

    def fire(j, sj):
        @pl.when(j >= 2)
        def _():
            _drain(bufh[sj], s_sh.at[pl.ds(0, B)], ssem[sj])

        eb = jnp.where(j == nmain, xeb, wbase + j * B)
        pltpu.sync_copy(dst_hbm.at[pl.ds(eb, B)], idx[sj])
        pltpu.async_copy(z3_hbm.at[pl.ds(eb // 2, B // 2)], bufz[sj],
                         rsem[sj])

    fire(0, 0)

    def halfiter(i, s, c):
        @pl.when(i < nmain)
        def _():
            fire(i + 1, 1 - s)

        _drain(z3_hbm.at[pl.ds(0, B // 2)], bufz[s], rsem[s])
        lax.fori_loop(0, B // 2, make_pair_body(bufz[s], bufh[s], None), 0)
        pltpu.async_copy(bufh[s], s_sh.at[idx[s]], ssem[s], add=True)
        return c

    def outer(k, c):
        c = halfiter(2 * k, 0, c)
        c = halfiter(2 * k + 1, 1, c)
        return c

    lax.fori_loop(0, nmain // 2, outer, 0)

    onevf = onev * flagf

    def obx(i, c):
        bufh0[i, pl.ds(64, L)] = onevf
        return c

    _drain(z3_hbm.at[pl.ds(0, B // 2)], bufz[0], rsem[0])
    lax.fori_loop(0, B, obx, 0)
    lax.fori_loop(0, B // 2, make_pair_body(bufz[0], bufh[0], flagf), 0)
    pltpu.async_copy(bufh[0], s_sh.at[idx[0]], ssem[0], add=True)

    _drain(bufh[0], s_sh.at[pl.ds(0, B)], ssem[0])
    _drain(bufh[1], s_sh.at[pl.ds(0, B)], ssem[1])

    plsc.subcore_barrier()

    ob2 = cid * NPAD + rb
    for r in range(rows_per_sub // B):
        pltpu.sync_copy(s_sh.at[pl.ds(rb + r * B, B)],
                        out_hbm.at[pl.ds(ob2 + r * B, B)])


def _k4(z3p, dst, scsh):
    return pl.kernel(
        _k4_body,
        out_type=jax.ShapeDtypeStruct((NC * NPAD, 80), _F32),
        mesh=_sc_mesh(),
        compiler_params=pltpu.CompilerParams(use_tc_tiling_on_sc=False),
        scratch_types=[
            pltpu.VMEM((B4,), jnp.int32),
            pltpu.VMEM((B4,), jnp.int32),
            pltpu.VMEM((B4 // 2, 128), _F32),
            pltpu.VMEM((B4 // 2, 128), _F32),
            pltpu.VMEM((B4, 80), _F32),
            pltpu.VMEM((B4, 80), _F32),
            pltpu.VMEM((256,), _F32),
            pltpu.VMEM_SHARED((NPAD, 80), _F32),
            pltpu.SemaphoreType.DMA,
            pltpu.SemaphoreType.DMA,
            pltpu.SemaphoreType.DMA,
            pltpu.SemaphoreType.DMA,
        ],
    )(z3p, dst, scsh)


def _k0_body(x_ref, w1_ref, t_ref):
    nf = x_ref.shape[1]
    x = x_ref[...]
    w1a = w1_ref[:nf]
    w1b = w1_ref[nf:]
    t_ref[:, :64] = jnp.dot(x, w1a - w1b, preferred_element_type=_F32)
    t_ref[:, 64:] = jnp.dot(x, w1b, preferred_element_type=_F32)


def _k0(x, w1):
    n = x.shape[0]
    return pl.pallas_call(
        _k0_body,
        out_shape=jax.ShapeDtypeStruct((n, 128), _F32),
    )(x, w1)


def _mlp_body(z_ref, w_ref, scsh_ref, zo_ref, st_ref, acc_ref):
    i = pl.program_id(0)
    h = jnp.maximum(z_ref[...] * scsh_ref[0:1, :] + scsh_ref[1:2, :], 0.0)
    z2 = jnp.dot(h, w_ref[...], preferred_element_type=_F32)
    zo_ref[...] = z2

    @pl.when(i == 0)
    def _():
        acc_ref[...] = jnp.zeros_like(acc_ref)

    acc_ref[0:1, :] += jnp.sum(z2, axis=0, keepdims=True)
    acc_ref[1:2, :] += jnp.sum(z2 * z2, axis=0, keepdims=True)

    @pl.when(i == pl.num_programs(0) - 1)
    def _():
        st_ref[...] = acc_ref[...]


def _mlp_layer(zp, wd, scshp, bp=2000):
    e2 = zp.shape[0]
    grid = e2 // bp
    return pl.pallas_call(
        _mlp_body,
        grid=(grid,),
        in_specs=[
            pl.BlockSpec((bp, 128), lambda i: (i, 0)),
            pl.BlockSpec((128, 128), lambda i: (0, 0)),
            pl.BlockSpec((2, 128), lambda i: (0, 0)),
        ],
        out_specs=[
            pl.BlockSpec((bp, 128), lambda i: (i, 0)),
            pl.BlockSpec((2, 128), lambda i: (0, 0)),
        ],
        out_shape=[jax.ShapeDtypeStruct((e2, 128), _F32),
                   jax.ShapeDtypeStruct((2, 128), _F32)],
        scratch_shapes=[pltpu.VMEM((2, 128), _F32)],
    )(zp, wd, scshp)


def _k5_body(sagg_ref, x_ref, ndw1_ref, ndg_ref, ndbe_ref, ndw2_ref,
             ndb2_ref, edw1_ref, nout_ref, c_ref):
    n = x_ref.shape[0]
    h = ndw1_ref.shape[0] - x_ref.shape[1]
    s = sagg_ref[0:n, 0:h] + sagg_ref[NPAD:NPAD + n, 0:h]
    cnt = sagg_ref[0:n, h:h + 1] + sagg_ref[NPAD:NPAD + n, h:h + 1]
    agg = s / jnp.maximum(cnt, 1.0)
    x = x_ref[...]
    zn = (jnp.dot(agg, ndw1_ref[:h], preferred_element_type=_F32)
          + jnp.dot(x, ndw1_ref[h:], preferred_element_type=_F32))
    m = jnp.mean(zn, axis=0, keepdims=True)
    v = jnp.mean(zn * zn, axis=0, keepdims=True) - m * m
    hn = jnp.maximum((zn - m) * lax.rsqrt(v + 1e-5) * ndg_ref[...]
                     + ndbe_ref[...], 0.0)
    logit = jnp.dot(hn, ndw2_ref[...], preferred_element_type=_F32) \
        + ndb2_ref[...]
    nout_ref[...] = 1.0 / (1.0 + jnp.exp(-logit))
    c_ref[...] = (jnp.dot(agg, edw1_ref[:h], preferred_element_type=_F32)
                  + jnp.dot(x, edw1_ref[h:], preferred_element_type=_F32))


def _k5(sagg, x, ndw1, ndg, ndbe, ndw2, ndb2, edw1):
    n = x.shape[0]
    mid = ndw1.shape[1]
    return pl.pallas_call(
        _k5_body,
        out_shape=[jax.ShapeDtypeStruct((n, 1), _F32),
                   jax.ShapeDtypeStruct((n, mid), _F32)],
    )(sagg, x, ndw1, ndg, ndbe, ndw2, ndb2, edw1)


def _k7_body(ze_ref, scsh_ref, w2_ref, b2_ref, out_ref):
    he = jnp.maximum(ze_ref[...] * scsh_ref[0:1, :] + scsh_ref[1:2, :], 0.0)
    logit = jnp.dot(he, w2_ref[...], preferred_element_type=_F32) \
        + b2_ref[...]
    out_ref[...] = 1.0 / (1.0 + jnp.exp(-logit))


def _k7(ze, scsh, w2, b2, be=4000):
    e, h = ze.shape
    grid = e // be
    return pl.pallas_call(
        _k7_body,
        grid=(grid,),
        in_specs=[
            pl.BlockSpec((be, h), lambda i: (i, 0)),
            pl.BlockSpec((2, h), lambda i: (0, 0)),
            pl.BlockSpec((h, 1), lambda i: (0, 0)),
            pl.BlockSpec((1, 1), lambda i: (0, 0)),
        ],
        out_specs=pl.BlockSpec((be, 1), lambda i: (i, 0)),
        out_shape=jax.ShapeDtypeStruct((e, 1), _F32),
    )(ze, scsh, w2, b2)


def _scale_shift(sums, sumsq, e, g, be):
    m = sums / e
    v = sumsq / e - m * m
    sc = g * lax.rsqrt(v + 1e-5)
    return sc, be - m * sc


def _blockdiag(w):
    h = w.shape[0]
    z = jnp.zeros((h, h), _F32)
    return jnp.block([[w, z], [z, w]])


def kernel(x, edge_index, ec_W1, ec_b1, ec_g1, ec_be1, ec_W2, ec_b2, ec_g2,
           ec_be2, ec_W3, ec_b3, ec_g3, ec_be3, nd_W1, nd_b1, nd_g1, nd_be1,
           nd_W2, nd_b2, ed_W1, ed_b1, ed_g1, ed_be1, ed_W2, ed_b2):
    del ec_b1, ec_b2, ec_b3, nd_b1, ed_b1
    e = edge_index.shape[1]
    ef = jnp.float32(e)
    dst = edge_index[1]

    t = _k0(x, ec_W1)
    z1p, st1 = _k1(t, edge_index)

    st1 = st1.reshape(NW, 128).sum(axis=0)
    sc1, sh1 = _scale_shift(st1[:64], st1[64:], ef, ec_g1, ec_be1)
    scsh1p = jnp.stack([jnp.tile(sc1, 2), jnp.tile(sh1, 2)])
    z2p, st2 = _mlp_layer(z1p, _blockdiag(ec_W2), scsh1p)

    sc2, sh2 = _scale_shift(st2[0, :64] + st2[0, 64:],
                            st2[1, :64] + st2[1, 64:], ef, ec_g2, ec_be2)
    scsh2p = jnp.stack([jnp.tile(sc2, 2), jnp.tile(sh2, 2)])
    z3p, st3 = _mlp_layer(z2p, _blockdiag(ec_W3), scsh2p)

    sc3, sh3 = _scale_shift(st3[0, :64] + st3[0, 64:],
                            st3[1, :64] + st3[1, 64:], ef, ec_g3, ec_be3)
    scsh3p = jnp.concatenate([jnp.tile(sc3, 2), jnp.tile(sh3, 2)])
    sagg = _k4(z3p, dst, scsh3p)

    node_out, c = _k5(sagg, x, nd_W1, nd_g1.reshape(1, -1),
                      nd_be1.reshape(1, -1), nd_W2, nd_b2.reshape(1, 1),
                      ed_W1)

    ze, ste = _k6(c, edge_index)
    ste = ste.reshape(NW, 256).sum(axis=0)
    sce, she = _scale_shift(ste[:128], ste[128:], ef, ed_g1, ed_be1)
    edge_out = _k7(ze, jnp.stack([sce, she]), ed_W2, ed_b2.reshape(1, 1))
    return (node_out, edge_out)


# speedup vs baseline: 4.1265x; 1.0276x over previous
import jax
import jax.numpy as jnp
from jax import lax
from jax.experimental import pallas as pl
from jax.experimental.pallas import tpu as pltpu
from jax.experimental.pallas import tpu_sc as plsc

NC = 2
NS = 16
NW = NC * NS
L = 16
EPB = 128
B4 = 128
NPAD = 10240

_F32 = jnp.float32


def _wid():
    return lax.axis_index("s") * NC + lax.axis_index("c")


def _sc_mesh():
    return plsc.VectorSubcoreMesh(core_axis_name="c", subcore_axis_name="s")


def _drain(src, dst, sem):
    pltpu.make_async_copy(src, dst, sem).wait()


def _k1_body(t_hbm, ei_hbm, z1_hbm, st_hbm,
             idx0, idx1, bufd0, bufd1, bufs0, bufs1,
             bufz0, bufz1, acc_v,
             gd0, gd1, gs0, gs1, w0, w1):
    wid = _wid()
    e = 4 * z1_hbm.shape[0]
    nb_total = e // EPB
    nmain = nb_total // NW
    nxw = nb_total % NW
    epw = nmain * EPB
    wbase = wid * epw
    zbase = wid * (epw // 4)
    has_x = wid < nxw
    xeb = jnp.where(has_x, NW * epw + wid * EPB, wbase)
    xzoff = jnp.where(has_x, NW * (epw // 4) + wid * (EPB // 4), zbase)
    flagf = jnp.where(has_x, jnp.full((L,), 1.0, _F32),
                      jnp.zeros((L,), _F32))
    idx = [idx0, idx1]
    bufd, bufs = [bufd0, bufd1], [bufs0, bufs1]
    bufz = [bufz0, bufz1]
    gsemd, gsems, wsem = [gd0, gd1], [gs0, gs1], [w0, w1]

    def make_pair_body(bd, bs, bz, scale):
        def pair_body(p, carry):
            out = list(carry)
            for qi in range(4):
                row = 4 * p + qi
                for g in range(4):
                    a = bd[row, pl.ds(g * L, L)]
                    b = bs[row, pl.ds(64 + g * L, L)]
                    z = a + b
                    bz[p, pl.ds(qi * 64 + g * L, L)] = z
                    zs = z if scale is None else z * scale
                    out[g] = out[g] + zs
                    out[4 + g] = out[4 + g] + z * zs
            return tuple(out)
        return pair_body

    def fire(j, sj):
        eb = jnp.where(j == nmain, xeb, wbase + j * EPB)
        pltpu.sync_copy(ei_hbm.at[:, pl.ds(eb, EPB)], idx[sj])
        pltpu.async_copy(t_hbm.at[idx[sj].at[1]], bufd[sj], gsemd[sj])
        pltpu.async_copy(t_hbm.at[idx[sj].at[0]], bufs[sj], gsems[sj])

    fire(0, 0)

    def halfiter(i, s, carry):
        @pl.when(i < nmain)
        def _():
            fire(i + 1, 1 - s)

        _drain(t_hbm.at[pl.ds(0, EPB)], bufd[s], gsemd[s])
        _drain(t_hbm.at[pl.ds(0, EPB)], bufs[s], gsems[s])

        @pl.when(i >= 2)
        def _():
            _drain(bufz[s], z1_hbm.at[pl.ds(0, EPB // 4)], wsem[s])

        carry = lax.fori_loop(
            0, EPB // 4, make_pair_body(bufd[s], bufs[s], bufz[s], None),
            carry)
        pltpu.async_copy(
            bufz[s], z1_hbm.at[pl.ds(zbase + i * (EPB // 4), EPB // 4)],
            wsem[s])
        return carry

    def outer(k, carry):
        carry = halfiter(2 * k, 0, carry)
        carry = halfiter(2 * k + 1, 1, carry)
        return carry

    zero = jnp.zeros((L,), _F32)
    carry = lax.fori_loop(0, nmain // 2, outer, (zero,) * 8)

    _drain(t_hbm.at[pl.ds(0, EPB)], bufd[0], gsemd[0])
    _drain(t_hbm.at[pl.ds(0, EPB)], bufs[0], gsems[0])
    _drain(bufz[0], z1_hbm.at[pl.ds(0, EPB // 4)], wsem[0])
    carry = lax.fori_loop(
        0, EPB // 4, make_pair_body(bufd[0], bufs[0], bufz[0], flagf), carry)
    pltpu.async_copy(bufz[0], z1_hbm.at[pl.ds(xzoff, EPB // 4)], wsem[0])

    _drain(bufz[0], z1_hbm.at[pl.ds(0, EPB // 4)], wsem[0])
    _drain(bufz[1], z1_hbm.at[pl.ds(0, EPB // 4)], wsem[1])

    for g in range(4):
        acc_v[pl.ds(g * L, L)] = carry[g]
        acc_v[pl.ds(64 + g * L, L)] = carry[4 + g]
    pltpu.sync_copy(acc_v, st_hbm.at[pl.ds(wid * 128, 128)])


def _k1(t, ei):
    e = ei.shape[1]
    return pl.kernel(
        _k1_body,
        out_type=[jax.ShapeDtypeStruct((e // 4, 256), _F32),
                  jax.ShapeDtypeStruct((NW * 128,), _F32)],
        mesh=_sc_mesh(),
        scratch_types=[
            pltpu.VMEM((2, EPB), jnp.int32),
            pltpu.VMEM((2, EPB), jnp.int32),
            pltpu.VMEM((EPB, 128), _F32),
            pltpu.VMEM((EPB, 128), _F32),
            pltpu.VMEM((EPB, 128), _F32),
            pltpu.VMEM((EPB, 128), _F32),
            pltpu.VMEM((EPB // 4, 256), _F32),
            pltpu.VMEM((EPB // 4, 256), _F32),
            pltpu.VMEM((128,), _F32),
            pltpu.SemaphoreType.DMA,
            pltpu.SemaphoreType.DMA,
            pltpu.SemaphoreType.DMA,
            pltpu.SemaphoreType.DMA,
            pltpu.SemaphoreType.DMA,
            pltpu.SemaphoreType.DMA,
        ],
    )(t, ei)


def _k6_body(c_hbm, ei_hbm, ze_hbm, st_hbm,
             idx0, idx1, bufd0, bufd1, bufs0, bufs1,
             bufz0, bufz1, acc_v,
             gd0, gd1, gs0, gs1, w0, w1):
    wid = _wid()
    e = ze_hbm.shape[0]
    nb_total = e // EPB
    nmain = nb_total // NW
    nxw = nb_total % NW
    epw = nmain * EPB
    wbase = wid * epw
    has_x = wid < nxw
    xeb = jnp.where(has_x, NW * epw + wid * EPB, wbase)
    flagf = jnp.where(has_x, jnp.full((L,), 1.0, _F32),
                      jnp.zeros((L,), _F32))
    idx = [idx0, idx1]
    bufd, bufs = [bufd0, bufd1], [bufs0, bufs1]
    bufz = [bufz0, bufz1]
    gsemd, gsems, wsem = [gd0, gd1], [gs0, gs1], [w0, w1]

    def make_edge_body(bd, bs, bz, scale):
        def edge_body(row, carry):
            out = list(carry)
            for g in range(8):
                sv = bs[row, pl.ds(g * L, L)]
                dv = bd[row, pl.ds(g * L, L)]
                z = sv - dv
                bz[row, pl.ds(g * L, L)] = z
                zs = z if scale is None else z * scale
                out[g] = out[g] + zs
                out[8 + g] = out[8 + g] + z * zs
            return tuple(out)
        return edge_body

    def fire(j, sj):
        eb = jnp.where(j == nmain, xeb, wbase + j * EPB)
        pltpu.sync_copy(ei_hbm.at[:, pl.ds(eb, EPB)], idx[sj])
        pltpu.async_copy(c_hbm.at[idx[sj].at[1]], bufd[sj], gsemd[sj])
        pltpu.async_copy(c_hbm.at[idx[sj].at[0]], bufs[sj], gsems[sj])

    fire(0, 0)

    def halfiter(i, s, carry):
        @pl.when(i < nmain)
        def _():
            fire(i + 1, 1 - s)

        _drain(c_hbm.at[pl.ds(0, EPB)], bufd[s], gsemd[s])
        _drain(c_hbm.at[pl.ds(0, EPB)], bufs[s], gsems[s])

        @pl.when(i >= 2)
        def _():
            _drain(bufz[s], ze_hbm.at[pl.ds(0, EPB)], wsem[s])

        carry = lax.fori_loop(
            0, EPB, make_edge_body(bufd[s], bufs[s], bufz[s], None), carry)
        pltpu.async_copy(bufz[s], ze_hbm.at[pl.ds(wbase + i * EPB, EPB)],
                         wsem[s])
        return carry

    def outer(k, carry):
        carry = halfiter(2 * k, 0, carry)
        carry = halfiter(2 * k + 1, 1, carry)
        return carry

    zero = jnp.zeros((L,), _F32)
    carry = lax.fori_loop(0, nmain // 2, outer, (zero,) * 16)

    _drain(c_hbm.at[pl.ds(0, EPB)], bufd[0], gsemd[0])
    _drain(c_hbm.at[pl.ds(0, EPB)], bufs[0], gsems[0])
    _drain(bufz[0], ze_hbm.at[pl.ds(0, EPB)], wsem[0])
    carry = lax.fori_loop(
        0, EPB, make_edge_body(bufd[0], bufs[0], bufz[0], flagf), carry)
    pltpu.async_copy(bufz[0], ze_hbm.at[pl.ds(xeb, EPB)], wsem[0])

    _drain(bufz[0], ze_hbm.at[pl.ds(0, EPB)], wsem[0])
    _drain(bufz[1], ze_hbm.at[pl.ds(0, EPB)], wsem[1])

    for g in range(8):
        acc_v[pl.ds(g * L, L)] = carry[g]
        acc_v[pl.ds(128 + g * L, L)] = carry[8 + g]
    pltpu.sync_copy(acc_v, st_hbm.at[pl.ds(wid * 256, 256)])


def _k6(c, ei):
    e = ei.shape[1]
    return pl.kernel(
        _k6_body,
        out_type=[jax.ShapeDtypeStruct((e, 128), _F32),
                  jax.ShapeDtypeStruct((NW * 256,), _F32)],
        mesh=_sc_mesh(),
        scratch_types=[
            pltpu.VMEM((2, EPB), jnp.int32),
            pltpu.VMEM((2, EPB), jnp.int32),
            pltpu.VMEM((EPB, 128), _F32),
            pltpu.VMEM((EPB, 128), _F32),
            pltpu.VMEM((EPB, 128), _F32),
            pltpu.VMEM((EPB, 128), _F32),
            pltpu.VMEM((EPB, 128), _F32),
            pltpu.VMEM((EPB, 128), _F32),
            pltpu.VMEM((256,), _F32),
            pltpu.SemaphoreType.DMA,
            pltpu.SemaphoreType.DMA,
            pltpu.SemaphoreType.DMA,
            pltpu.SemaphoreType.DMA,
            pltpu.SemaphoreType.DMA,
            pltpu.SemaphoreType.DMA,
        ],
    )(c, ei)


def _k4_body(z3_hbm, dst_hbm, scsh_hbm, out_hbm,
             idx0, idx1, bufz0, bufz1, bufh0, bufh1, scsh_v, s_sh,
             r0, r1, sc0, sc1):
    cid = lax.axis_index("c")
    sid = lax.axis_index("s")
    wid = sid * NC + cid
    B = B4
    e = 4 * z3_hbm.shape[0]
    nb_total = e // B
    nmain = nb_total // NW
    nxw = nb_total % NW
    epw = nmain * B
    wbase = wid * epw
    has_x = wid < nxw
    xeb = jnp.where(has_x, NW * epw + wid * B, wbase)
    flagf = jnp.where(has_x, jnp.full((L,), 1.0, _F32),
                      jnp.zeros((L,), _F32))
    rows_per_sub = NPAD // NS
    idx = [idx0, idx1]
    bufz = [bufz0, bufz1]
    bufh = [bufh0, bufh1]
    rsem, ssem = [r0, r1], [sc0, sc1]

    pltpu.sync_copy(scsh_hbm, scsh_v)

    zero = jnp.zeros((L,), _F32)

    def zb(i, c):
        for g in range(5):
            bufh0[i, pl.ds(g * L, L)] = zero
            bufh1[i, pl.ds(g * L, L)] = zero
        return c

    lax.fori_loop(0, B, zb, 0)
    rb = sid * rows_per_sub
    for r in range(rows_per_sub // B):
        pltpu.sync_copy(bufh0, s_sh.at[pl.ds(rb + r * B, B)])

    onev = jnp.where(lax.iota(jnp.int32, L) == 0,
                     jnp.full((L,), 1.0, _F32), zero)

    def ob(i, c):
        bufh0[i, pl.ds(64, L)] = onev
        bufh1[i, pl.ds(64, L)] = onev
        return c

    lax.fori_loop(0, B, ob, 0)
    plsc.subcore_barrier()

    scv = [scsh_v[pl.ds(c * L, L)] for c in range(16)]
    shv = [scsh_v[pl.ds(256 + c * L, L)] for c in range(16)]

    def make_pair_body(bz, bh, scale):
        def pair_body(p, c):
            for qi in range(4):
                row = 4 * p + qi
                for g in range(4):
                    z = bz[p, pl.ds(qi * 64 + g * L, L)]
                    h = jnp.maximum(
                        z * scv[4 * qi + g] + shv[4 * qi + g], 0.0)
                    if scale is not None:
                        h = h * scale
                    bh[row, pl.ds(g * L, L)] = h
            return c
        return pair_body

    def fire(j, sj):
        @pl.when(j >= 2)
        def _():
            _drain(bufh[sj], s_sh.at[pl.ds(0, B)], ssem[sj])

        eb = jnp.where(j == nmain, xeb, wbase + j * B)
        pltpu.sync_copy(dst_hbm.at[pl.ds(eb, B)], idx[sj])
        pltpu.async_copy(z3_hbm.at[pl.ds(eb // 4, B // 4)], bufz[sj],
                         rsem[sj])

    fire(0, 0)

    def halfiter(i, s, c):
        @pl.when(i < nmain)
        def _():
            fire(i + 1, 1 - s)

        _drain(z3_hbm.at[pl.ds(0, B // 4)], bufz[s], rsem[s])
        lax.fori_loop(0, B // 4, make_pair_body(bufz[s], bufh[s], None), 0)
        pltpu.async_copy(bufh[s], s_sh.at[idx[s]], ssem[s], add=True)
        return c

    def outer(k, c):
        c = halfiter(2 * k, 0, c)
        c = halfiter(2 * k + 1, 1, c)
        return c

    lax.fori_loop(0, nmain // 2, outer, 0)

    onevf = onev * flagf

    def obx(i, c):
        bufh0[i, pl.ds(64, L)] = onevf
        return c

    _drain(z3_hbm.at[pl.ds(0, B // 4)], bufz[0], rsem[0])
    lax.fori_loop(0, B, obx, 0)
    lax.fori_loop(0, B // 4, make_pair_body(bufz[0], bufh[0], flagf), 0)
    pltpu.async_copy(bufh[0], s_sh.at[idx[0]], ssem[0], add=True)

    _drain(bufh[0], s_sh.at[pl.ds(0, B)], ssem[0])
    _drain(bufh[1], s_sh.at[pl.ds(0, B)], ssem[1])

    plsc.subcore_barrier()

    ob2 = cid * NPAD + rb
    for r in range(rows_per_sub // B):
        pltpu.sync_copy(s_sh.at[pl.ds(rb + r * B, B)],
                        out_hbm.at[pl.ds(ob2 + r * B, B)])


def _k4(z3p, dst, scsh):
    return pl.kernel(
        _k4_body,
        out_type=jax.ShapeDtypeStruct((NC * NPAD, 80), _F32),
        mesh=_sc_mesh(),
        compiler_params=pltpu.CompilerParams(use_tc_tiling_on_sc=False),
        scratch_types=[
            pltpu.VMEM((B4,), jnp.int32),
            pltpu.VMEM((B4,), jnp.int32),
            pltpu.VMEM((B4 // 4, 256), _F32),
            pltpu.VMEM((B4 // 4, 256), _F32),
            pltpu.VMEM((B4, 80), _F32),
            pltpu.VMEM((B4, 80), _F32),
            pltpu.VMEM((512,), _F32),
            pltpu.VMEM_SHARED((NPAD, 80), _F32),
            pltpu.SemaphoreType.DMA,
            pltpu.SemaphoreType.DMA,
            pltpu.SemaphoreType.DMA,
            pltpu.SemaphoreType.DMA,
        ],
    )(z3p, dst, scsh)


def _k0_body(x_ref, w1_ref, t_ref):
    nf = x_ref.shape[1]
    x = x_ref[...]
    w1a = w1_ref[:nf]
    w1b = w1_ref[nf:]
    t_ref[:, :64] = jnp.dot(x, w1a - w1b, preferred_element_type=_F32)
    t_ref[:, 64:] = jnp.dot(x, w1b, preferred_element_type=_F32)


def _k0(x, w1):
    n = x.shape[0]
    return pl.pallas_call(
        _k0_body,
        out_shape=jax.ShapeDtypeStruct((n, 128), _F32),
    )(x, w1)


def _mlp_body(z_ref, w_ref, scsh_ref, zo_ref, st_ref, acc_ref):
    i = pl.program_id(0)
    h = jnp.maximum(z_ref[...] * scsh_ref[0:1, :] + scsh_ref[1:2, :], 0.0)
    z2 = jnp.dot(h, w_ref[...], preferred_element_type=_F32)
    zo_ref[...] = z2

    @pl.when(i == 0)
    def _():
        acc_ref[...] = jnp.zeros_like(acc_ref)

    acc_ref[0:1, :] += jnp.sum(z2, axis=0, keepdims=True)
    acc_ref[1:2, :] += jnp.sum(z2 * z2, axis=0, keepdims=True)

    @pl.when(i == pl.num_programs(0) - 1)
    def _():
        st_ref[...] = acc_ref[...]


def _mlp_layer(zp, wd, scshp, bp=2000):
    e4 = zp.shape[0]
    grid = e4 // bp
    return pl.pallas_call(
        _mlp_body,
        grid=(grid,),
        in_specs=[
            pl.BlockSpec((bp, 256), lambda i: (i, 0)),
            pl.BlockSpec((256, 256), lambda i: (0, 0)),
            pl.BlockSpec((2, 256), lambda i: (0, 0)),
        ],
        out_specs=[
            pl.BlockSpec((bp, 256), lambda i: (i, 0)),
            pl.BlockSpec((2, 256), lambda i: (0, 0)),
        ],
        out_shape=[jax.ShapeDtypeStruct((e4, 256), _F32),
                   jax.ShapeDtypeStruct((2, 256), _F32)],
        scratch_shapes=[pltpu.VMEM((2, 256), _F32)],
    )(zp, wd, scshp)


def _k5_body(sagg_ref, x_ref, ndw1_ref, ndg_ref, ndbe_ref, ndw2_ref,
             ndb2_ref, edw1_ref, nout_ref, c_ref):
    n = x_ref.shape[0]
    h = ndw1_ref.shape[0] - x_ref.shape[1]
    s = sagg_ref[0:n, 0:h] + sagg_ref[NPAD:NPAD + n, 0:h]
    cnt = sagg_ref[0:n, h:h + 1] + sagg_ref[NPAD:NPAD + n, h:h + 1]
    agg = s / jnp.maximum(cnt, 1.0)
    x = x_ref[...]
    zn = (jnp.dot(agg, ndw1_ref[:h], preferred_element_type=_F32)
          + jnp.dot(x, ndw1_ref[h:], preferred_element_type=_F32))
    m = jnp.mean(zn, axis=0, keepdims=True)
    v = jnp.mean(zn * zn, axis=0, keepdims=True) - m * m
    hn = jnp.maximum((zn - m) * lax.rsqrt(v + 1e-5) * ndg_ref[...]
                     + ndbe_ref[...], 0.0)
    logit = jnp.dot(hn, ndw2_ref[...], preferred_element_type=_F32) \
        + ndb2_ref[...]
    nout_ref[...] = 1.0 / (1.0 + jnp.exp(-logit))
    c_ref[...] = (jnp.dot(agg, edw1_ref[:h], preferred_element_type=_F32)
                  + jnp.dot(x, edw1_ref[h:], preferred_element_type=_F32))


def _k5(sagg, x, ndw1, ndg, ndbe, ndw2, ndb2, edw1):
    n = x.shape[0]
    mid = ndw1.shape[1]
    return pl.pallas_call(
        _k5_body,
        out_shape=[jax.ShapeDtypeStruct((n, 1), _F32),
                   jax.ShapeDtypeStruct((n, mid), _F32)],
    )(sagg, x, ndw1, ndg, ndbe, ndw2, ndb2, edw1)


def _k7_body(ze_ref, scsh_ref, w2_ref, b2_ref, out_ref):
    he = jnp.maximum(ze_ref[...] * scsh_ref[0:1, :] + scsh_ref[1:2, :], 0.0)
    logit = jnp.dot(he, w2_ref[...], preferred_element_type=_F32) \
        + b2_ref[...]
    out_ref[...] = 1.0 / (1.0 + jnp.exp(-logit))


def _k7(ze, scsh, w2, b2, be=4000):
    e, h = ze.shape
    grid = e // be
    return pl.pallas_call(
        _k7_body,
        grid=(grid,),
        in_specs=[
            pl.BlockSpec((be, h), lambda i: (i, 0)),
            pl.BlockSpec((2, h), lambda i: (0, 0)),
            pl.BlockSpec((h, 1), lambda i: (0, 0)),
            pl.BlockSpec((1, 1), lambda i: (0, 0)),
        ],
        out_specs=pl.BlockSpec((be, 1), lambda i: (i, 0)),
        out_shape=jax.ShapeDtypeStruct((e, 1), _F32),
    )(ze, scsh, w2, b2)


def _scale_shift(sums, sumsq, e, g, be):
    m = sums / e
    v = sumsq / e - m * m
    sc = g * lax.rsqrt(v + 1e-5)
    return sc, be - m * sc


def _blockdiag4(w):
    h = w.shape[0]
    z = jnp.zeros((h, h), _F32)
    return jnp.block([[w, z, z, z], [z, w, z, z], [z, z, w, z],
                      [z, z, z, w]])


def _fold4(row):
    return row.reshape(4, 64).sum(axis=0)


def kernel(x, edge_index, ec_W1, ec_b1, ec_g1, ec_be1, ec_W2, ec_b2, ec_g2,
           ec_be2, ec_W3, ec_b3, ec_g3, ec_be3, nd_W1, nd_b1, nd_g1, nd_be1,
           nd_W2, nd_b2, ed_W1, ed_b1, ed_g1, ed_be1, ed_W2, ed_b2):
    del ec_b1, ec_b2, ec_b3, nd_b1, ed_b1
    e = edge_index.shape[1]
    ef = jnp.float32(e)
    dst = edge_index[1]

    t = _k0(x, ec_W1)
    z1p, st1 = _k1(t, edge_index)

    st1 = st1.reshape(NW, 128).sum(axis=0)
    sc1, sh1 = _scale_shift(st1[:64], st1[64:], ef, ec_g1, ec_be1)
    scsh1p = jnp.stack([jnp.tile(sc1, 4), jnp.tile(sh1, 4)])
    z2p, st2 = _mlp_layer(z1p, _blockdiag4(ec_W2), scsh1p)

    sc2, sh2 = _scale_shift(_fold4(st2[0]), _fold4(st2[1]), ef, ec_g2,
                            ec_be2)
    scsh2p = jnp.stack([jnp.tile(sc2, 4), jnp.tile(sh2, 4)])
    z3p, st3 = _mlp_layer(z2p, _blockdiag4(ec_W3), scsh2p)

    sc3, sh3 = _scale_shift(_fold4(st3[0]), _fold4(st3[1]), ef, ec_g3,
                            ec_be3)
    scsh3p = jnp.concatenate([jnp.tile(sc3, 4), jnp.tile(sh3, 4)])
    sagg = _k4(z3p, dst, scsh3p)

    node_out, c = _k5(sagg, x, nd_W1, nd_g1.reshape(1, -1),
                      nd_be1.reshape(1, -1), nd_W2, nd_b2.reshape(1, 1),
                      ed_W1)

    ze, ste = _k6(c, edge_index)
    ste = ste.reshape(NW, 256).sum(axis=0)
    sce, she = _scale_shift(ste[:128], ste[128:], ef, ed_g1, ed_be1)
    edge_out = _k7(ze, jnp.stack([sce, she]), ed_W2, ed_b2.reshape(1, 1))
    return (node_out, edge_out)


# speedup vs baseline: 4.3111x; 1.0447x over previous
import jax
import jax.numpy as jnp
from jax import lax
from jax.experimental import pallas as pl
from jax.experimental.pallas import tpu as pltpu
from jax.experimental.pallas import tpu_sc as plsc

NC = 2
NS = 16
NW = NC * NS
L = 16
EPB = 128
B4 = 128
NPAD = 10240

_F32 = jnp.float32


def _wid():
    return lax.axis_index("s") * NC + lax.axis_index("c")


def _sc_mesh():
    return plsc.VectorSubcoreMesh(core_axis_name="c", subcore_axis_name="s")


def _drain(src, dst, sem):
    pltpu.make_async_copy(src, dst, sem).wait()


def _k1_body(t_hbm, ei_hbm, z1_hbm, st_hbm,
             idx0, idx1, bufd0, bufd1, bufs0, bufs1,
             bufz0, bufz1, acc_v,
             gd0, gd1, gs0, gs1, w0, w1, i0, i1):
    wid = _wid()
    e = 4 * z1_hbm.shape[0]
    nb_total = e // EPB
    nmain = nb_total // NW
    nxw = nb_total % NW
    epw = nmain * EPB
    wbase = wid * epw
    zbase = wid * (epw // 4)
    has_x = wid < nxw
    xeb = jnp.where(has_x, NW * epw + wid * EPB, wbase)
    xzoff = jnp.where(has_x, NW * (epw // 4) + wid * (EPB // 4), zbase)
    flagf = jnp.where(has_x, jnp.full((L,), 1.0, _F32),
                      jnp.zeros((L,), _F32))
    idx = [idx0, idx1]
    bufd, bufs = [bufd0, bufd1], [bufs0, bufs1]
    bufz = [bufz0, bufz1]
    gsemd, gsems, wsem = [gd0, gd1], [gs0, gs1], [w0, w1]
    isem = [i0, i1]

    def make_pair_body(bd, bs, bz, scale):
        def pair_body(p, carry):
            out = list(carry)
            for qi in range(4):
                row = 4 * p + qi
                for g in range(4):
                    a = bd[row, pl.ds(g * L, L)]
                    b = bs[row, pl.ds(64 + g * L, L)]
                    z = a + b
                    bz[p, pl.ds(qi * 64 + g * L, L)] = z
                    zs = z if scale is None else z * scale
                    out[g] = out[g] + zs
                    out[4 + g] = out[4 + g] + z * zs
            return tuple(out)
        return pair_body

    def fire_idx(j, sj):
        eb = jnp.where(j == nmain, xeb, wbase + j * EPB)
        pltpu.async_copy(ei_hbm.at[:, pl.ds(eb, EPB)], idx[sj], isem[sj])

    def fire_gather(j, sj):
        del j
        _drain(ei_hbm.at[:, pl.ds(0, EPB)], idx[sj], isem[sj])
        pltpu.async_copy(t_hbm.at[idx[sj].at[1]], bufd[sj], gsemd[sj])
        pltpu.async_copy(t_hbm.at[idx[sj].at[0]], bufs[sj], gsems[sj])

    fire_idx(0, 0)
    fire_gather(0, 0)
    fire_idx(1, 1)

    def halfiter(i, s, carry):
        @pl.when(i < nmain)
        def _():
            fire_gather(i + 1, 1 - s)

        _drain(t_hbm.at[pl.ds(0, EPB)], bufd[s], gsemd[s])
        _drain(t_hbm.at[pl.ds(0, EPB)], bufs[s], gsems[s])

        @pl.when(i + 2 <= nmain)
        def _():
            fire_idx(i + 2, s)

        @pl.when(i >= 2)
        def _():
            _drain(bufz[s], z1_hbm.at[pl.ds(0, EPB // 4)], wsem[s])

        carry = lax.fori_loop(
            0, EPB // 4, make_pair_body(bufd[s], bufs[s], bufz[s], None),
            carry)
        pltpu.async_copy(
            bufz[s], z1_hbm.at[pl.ds(zbase + i * (EPB // 4), EPB // 4)],
            wsem[s])
        return carry

    def outer(k, carry):
        carry = halfiter(2 * k, 0, carry)
        carry = halfiter(2 * k + 1, 1, carry)
        return carry

    zero = jnp.zeros((L,), _F32)
    carry = lax.fori_loop(0, nmain // 2, outer, (zero,) * 8)

    _drain(t_hbm.at[pl.ds(0, EPB)], bufd[0], gsemd[0])
    _drain(t_hbm.at[pl.ds(0, EPB)], bufs[0], gsems[0])
    _drain(bufz[0], z1_hbm.at[pl.ds(0, EPB // 4)], wsem[0])
    carry = lax.fori_loop(
        0, EPB // 4, make_pair_body(bufd[0], bufs[0], bufz[0], flagf), carry)
    pltpu.async_copy(bufz[0], z1_hbm.at[pl.ds(xzoff, EPB // 4)], wsem[0])

    _drain(bufz[0], z1_hbm.at[pl.ds(0, EPB // 4)], wsem[0])
    _drain(bufz[1], z1_hbm.at[pl.ds(0, EPB // 4)], wsem[1])

    for g in range(4):
        acc_v[pl.ds(g * L, L)] = carry[g]
        acc_v[pl.ds(64 + g * L, L)] = carry[4 + g]
    pltpu.sync_copy(acc_v, st_hbm.at[pl.ds(wid * 128, 128)])


def _k1(t, ei):
    e = ei.shape[1]
    return pl.kernel(
        _k1_body,
        out_type=[jax.ShapeDtypeStruct((e // 4, 256), _F32),
                  jax.ShapeDtypeStruct((NW * 128,), _F32)],
        mesh=_sc_mesh(),
        scratch_types=[
            pltpu.VMEM((2, EPB), jnp.int32),
            pltpu.VMEM((2, EPB), jnp.int32),
            pltpu.VMEM((EPB, 128), _F32),
            pltpu.VMEM((EPB, 128), _F32),
            pltpu.VMEM((EPB, 128), _F32),
            pltpu.VMEM((EPB, 128), _F32),
            pltpu.VMEM((EPB // 4, 256), _F32),
            pltpu.VMEM((EPB // 4, 256), _F32),
            pltpu.VMEM((128,), _F32),
            pltpu.SemaphoreType.DMA,
            pltpu.SemaphoreType.DMA,
            pltpu.SemaphoreType.DMA,
            pltpu.SemaphoreType.DMA,
            pltpu.SemaphoreType.DMA,
            pltpu.SemaphoreType.DMA,
            pltpu.SemaphoreType.DMA,
            pltpu.SemaphoreType.DMA,
        ],
    )(t, ei)


def _k6_body(c_hbm, ei_hbm, ze_hbm, st_hbm,
             idx0, idx1, bufd0, bufd1, bufs0, bufs1,
             bufz0, bufz1, acc_v,
             gd0, gd1, gs0, gs1, w0, w1, i0, i1):
    wid = _wid()
    e = ze_hbm.shape[0]
    nb_total = e // EPB
    nmain = nb_total // NW
    nxw = nb_total % NW
    epw = nmain * EPB
    wbase = wid * epw
    has_x = wid < nxw
    xeb = jnp.where(has_x, NW * epw + wid * EPB, wbase)
    flagf = jnp.where(has_x, jnp.full((L,), 1.0, _F32),
                      jnp.zeros((L,), _F32))
    idx = [idx0, idx1]
    bufd, bufs = [bufd0, bufd1], [bufs0, bufs1]
    bufz = [bufz0, bufz1]
    gsemd, gsems, wsem = [gd0, gd1], [gs0, gs1], [w0, w1]
    isem = [i0, i1]

    def make_edge_body(bd, bs, bz, scale):
        def edge_body(row, carry):
            out = list(carry)
            for g in range(8):
                sv = bs[row, pl.ds(g * L, L)]
                dv = bd[row, pl.ds(g * L, L)]
                z = sv - dv
                bz[row, pl.ds(g * L, L)] = z
                zs = z if scale is None else z * scale
                out[g] = out[g] + zs
                out[8 + g] = out[8 + g] + z * zs
            return tuple(out)
        return edge_body

    def fire_idx(j, sj):
        eb = jnp.where(j == nmain, xeb, wbase + j * EPB)
        pltpu.async_copy(ei_hbm.at[:, pl.ds(eb, EPB)], idx[sj], isem[sj])

    def fire_gather(j, sj):
        del j
        _drain(ei_hbm.at[:, pl.ds(0, EPB)], idx[sj], isem[sj])
        pltpu.async_copy(c_hbm.at[idx[sj].at[1]], bufd[sj], gsemd[sj])
        pltpu.async_copy(c_hbm.at[idx[sj].at[0]], bufs[sj], gsems[sj])

    fire_idx(0, 0)
    fire_gather(0, 0)
    fire_idx(1, 1)

    def halfiter(i, s, carry):
        @pl.when(i < nmain)
        def _():
            fire_gather(i + 1, 1 - s)

        _drain(c_hbm.at[pl.ds(0, EPB)], bufd[s], gsemd[s])
        _drain(c_hbm.at[pl.ds(0, EPB)], bufs[s], gsems[s])

        @pl.when(i + 2 <= nmain)
        def _():
            fire_idx(i + 2, s)

        @pl.when(i >= 2)
        def _():
            _drain(bufz[s], ze_hbm.at[pl.ds(0, EPB)], wsem[s])

        carry = lax.fori_loop(
            0, EPB, make_edge_body(bufd[s], bufs[s], bufz[s], None), carry)
        pltpu.async_copy(bufz[s], ze_hbm.at[pl.ds(wbase + i * EPB, EPB)],
                         wsem[s])
        return carry

    def outer(k, carry):
        carry = halfiter(2 * k, 0, carry)
        carry = halfiter(2 * k + 1, 1, carry)
        return carry

    zero = jnp.zeros((L,), _F32)
    carry = lax.fori_loop(0, nmain // 2, outer, (zero,) * 16)

    _drain(c_hbm.at[pl.ds(0, EPB)], bufd[0], gsemd[0])
    _drain(c_hbm.at[pl.ds(0, EPB)], bufs[0], gsems[0])
    _drain(bufz[0], ze_hbm.at[pl.ds(0, EPB)], wsem[0])
    carry = lax.fori_loop(
        0, EPB, make_edge_body(bufd[0], bufs[0], bufz[0], flagf), carry)
    pltpu.async_copy(bufz[0], ze_hbm.at[pl.ds(xeb, EPB)], wsem[0])

    _drain(bufz[0], ze_hbm.at[pl.ds(0, EPB)], wsem[0])
    _drain(bufz[1], ze_hbm.at[pl.ds(0, EPB)], wsem[1])

    for g in range(8):
        acc_v[pl.ds(g * L, L)] = carry[g]
        acc_v[pl.ds(128 + g * L, L)] = carry[8 + g]
    pltpu.sync_copy(acc_v, st_hbm.at[pl.ds(wid * 256, 256)])


def _k6(c, ei):
    e = ei.shape[1]
    return pl.kernel(
        _k6_body,
        out_type=[jax.ShapeDtypeStruct((e, 128), _F32),
                  jax.ShapeDtypeStruct((NW * 256,), _F32)],
        mesh=_sc_mesh(),
        scratch_types=[
            pltpu.VMEM((2, EPB), jnp.int32),
            pltpu.VMEM((2, EPB), jnp.int32),
            pltpu.VMEM((EPB, 128), _F32),
            pltpu.VMEM((EPB, 128), _F32),
            pltpu.VMEM((EPB, 128), _F32),
            pltpu.VMEM((EPB, 128), _F32),
            pltpu.VMEM((EPB, 128), _F32),
            pltpu.VMEM((EPB, 128), _F32),
            pltpu.VMEM((256,), _F32),
            pltpu.SemaphoreType.DMA,
            pltpu.SemaphoreType.DMA,
            pltpu.SemaphoreType.DMA,
            pltpu.SemaphoreType.DMA,
            pltpu.SemaphoreType.DMA,
            pltpu.SemaphoreType.DMA,
            pltpu.SemaphoreType.DMA,
            pltpu.SemaphoreType.DMA,
        ],
    )(c, ei)


def _k4_body(z3_hbm, dst_hbm, scsh_hbm, out_hbm,
             idx0, idx1, bufz0, bufz1, bufh0, bufh1, scsh_v, s_sh,
             r0, r1, sc0, sc1):
    cid = lax.axis_index("c")
    sid = lax.axis_index("s")
    wid = sid * NC + cid
    B = B4
    e = 4 * z3_hbm.shape[0]
    nb_total = e // B
    nmain = nb_total // NW
    nxw = nb_total % NW
    epw = nmain * B
    wbase = wid * epw
    has_x = wid < nxw
    xeb = jnp.where(has_x, NW * epw + wid * B, wbase)
    flagf = jnp.where(has_x, jnp.full((L,), 1.0, _F32),
                      jnp.zeros((L,), _F32))
    rows_per_sub = NPAD // NS
    idx = [idx0, idx1]
    bufz = [bufz0, bufz1]
    bufh = [bufh0, bufh1]
    rsem, ssem = [r0, r1], [sc0, sc1]

    pltpu.sync_copy(scsh_hbm, scsh_v)

    zero = jnp.zeros((L,), _F32)

    def zb(i, c):
        for g in range(5):
            bufh0[i, pl.ds(g * L, L)] = zero
            bufh1[i, pl.ds(g * L, L)] = zero
        return c

    lax.fori_loop(0, B, zb, 0)
    rb = sid * rows_per_sub
    for r in range(rows_per_sub // B):
        pltpu.sync_copy(bufh0, s_sh.at[pl.ds(rb + r * B, B)])

    onev = jnp.where(lax.iota(jnp.int32, L) == 0,
                     jnp.full((L,), 1.0, _F32), zero)

    def ob(i, c):
        bufh0[i, pl.ds(64, L)] = onev
        bufh1[i, pl.ds(64, L)] = onev
        return c

    lax.fori_loop(0, B, ob, 0)
    plsc.subcore_barrier()

    scv = [scsh_v[pl.ds(c * L, L)] for c in range(16)]
    shv = [scsh_v[pl.ds(256 + c * L, L)] for c in range(16)]

    def make_pair_body(bz, bh, scale):
        def pair_body(p, c):
            for qi in range(4):
                row = 4 * p + qi
                for g in range(4):
                    z = bz[p, pl.ds(qi * 64 + g * L, L)]
                    h = jnp.maximum(
                        z * scv[4 * qi + g] + shv[4 * qi + g], 0.0)
                    if scale is not None:
                        h = h * scale
                    bh[row, pl.ds(g * L, L)] = h
            return c
        return pair_body

    def fire(j, sj):
        @pl.when(j >= 2)
        def _():
            _drain(bufh[sj], s_sh.at[pl.ds(0, B)], ssem[sj])

        eb = jnp.where(j == nmain, xeb, wbase + j * B)
        pltpu.sync_copy(dst_hbm.at[pl.ds(eb, B)], idx[sj])
        pltpu.async_copy(z3_hbm.at[pl.ds(eb // 4, B // 4)], bufz[sj],
                         rsem[sj])

    fire(0, 0)

    def halfiter(i, s, c):
        @pl.when(i < nmain)
        def _():
            fire(i + 1, 1 - s)

        _drain(z3_hbm.at[pl.ds(0, B // 4)], bufz[s], rsem[s])
        lax.fori_loop(0, B // 4, make_pair_body(bufz[s], bufh[s], None), 0)
        pltpu.async_copy(bufh[s], s_sh.at[idx[s]], ssem[s], add=True)
        return c

    def outer(k, c):
        c = halfiter(2 * k, 0, c)
        c = halfiter(2 * k + 1, 1, c)
        return c

    lax.fori_loop(0, nmain // 2, outer, 0)

    onevf = onev * flagf

    def obx(i, c):
        bufh0[i, pl.ds(64, L)] = onevf
        return c

    _drain(z3_hbm.at[pl.ds(0, B // 4)], bufz[0], rsem[0])
    lax.fori_loop(0, B, obx, 0)
    lax.fori_loop(0, B // 4, make_pair_body(bufz[0], bufh[0], flagf), 0)
    pltpu.async_copy(bufh[0], s_sh.at[idx[0]], ssem[0], add=True)

    _drain(bufh[0], s_sh.at[pl.ds(0, B)], ssem[0])
    _drain(bufh[1], s_sh.at[pl.ds(0, B)], ssem[1])

    plsc.subcore_barrier()

    ob2 = cid * NPAD + rb
    for r in range(rows_per_sub // B):
        pltpu.sync_copy(s_sh.at[pl.ds(rb + r * B, B)],
                        out_hbm.at[pl.ds(ob2 + r * B, B)])


def _k4(z3p, dst, scsh):
    return pl.kernel(
        _k4_body,
        out_type=jax.ShapeDtypeStruct((NC * NPAD, 80), _F32),
        mesh=_sc_mesh(),
        compiler_params=pltpu.CompilerParams(use_tc_tiling_on_sc=False),
        scratch_types=[
            pltpu.VMEM((B4,), jnp.int32),
            pltpu.VMEM((B4,), jnp.int32),
            pltpu.VMEM((B4 // 4, 256), _F32),
            pltpu.VMEM((B4 // 4, 256), _F32),
            pltpu.VMEM((B4, 80), _F32),
            pltpu.VMEM((B4, 80), _F32),
            pltpu.VMEM((512,), _F32),
            pltpu.VMEM_SHARED((NPAD, 80), _F32),
            pltpu.SemaphoreType.DMA,
            pltpu.SemaphoreType.DMA,
            pltpu.SemaphoreType.DMA,
            pltpu.SemaphoreType.DMA,
        ],
    )(z3p, dst, scsh)


def _k0_body(x_ref, w1_ref, t_ref):
    nf = x_ref.shape[1]
    x = x_ref[...]
    w1a = w1_ref[:nf]
    w1b = w1_ref[nf:]
    t_ref[:, :64] = jnp.dot(x, w1a - w1b, preferred_element_type=_F32)
    t_ref[:, 64:] = jnp.dot(x, w1b, preferred_element_type=_F32)


def _k0(x, w1):
    n = x.shape[0]
    return pl.pallas_call(
        _k0_body,
        out_shape=jax.ShapeDtypeStruct((n, 128), _F32),
    )(x, w1)


def _mlp_body(z_ref, w_ref, scsh_ref, zo_ref, st_ref, acc_ref):
    i = pl.program_id(0)
    h = jnp.maximum(z_ref[...] * scsh_ref[0:1, :] + scsh_ref[1:2, :], 0.0)
    z2 = jnp.dot(h, w_ref[...], preferred_element_type=_F32)
    zo_ref[...] = z2

    @pl.when(i == 0)
    def _():
        acc_ref[...] = jnp.zeros_like(acc_ref)

    acc_ref[0:1, :] += jnp.sum(z2, axis=0, keepdims=True)
    acc_ref[1:2, :] += jnp.sum(z2 * z2, axis=0, keepdims=True)

    @pl.when(i == pl.num_programs(0) - 1)
    def _():
        st_ref[...] = acc_ref[...]


def _mlp_layer(zp, wd, scshp, bp=2000):
    e4 = zp.shape[0]
    grid = e4 // bp
    return pl.pallas_call(
        _mlp_body,
        grid=(grid,),
        in_specs=[
            pl.BlockSpec((bp, 256), lambda i: (i, 0)),
            pl.BlockSpec((256, 256), lambda i: (0, 0)),
            pl.BlockSpec((2, 256), lambda i: (0, 0)),
        ],
        out_specs=[
            pl.BlockSpec((bp, 256), lambda i: (i, 0)),
            pl.BlockSpec((2, 256), lambda i: (0, 0)),
        ],
        out_shape=[jax.ShapeDtypeStruct((e4, 256), _F32),
                   jax.ShapeDtypeStruct((2, 256), _F32)],
        scratch_shapes=[pltpu.VMEM((2, 256), _F32)],
    )(zp, wd, scshp)


def _k5_body(sagg_ref, x_ref, ndw1_ref, ndg_ref, ndbe_ref, ndw2_ref,
             ndb2_ref, edw1_ref, nout_ref, c_ref):
    n = x_ref.shape[0]
    h = ndw1_ref.shape[0] - x_ref.shape[1]
    s = sagg_ref[0:n, 0:h] + sagg_ref[NPAD:NPAD + n, 0:h]
    cnt = sagg_ref[0:n, h:h + 1] + sagg_ref[NPAD:NPAD + n, h:h + 1]
    agg = s / jnp.maximum(cnt, 1.0)
    x = x_ref[...]
    zn = (jnp.dot(agg, ndw1_ref[:h], preferred_element_type=_F32)
          + jnp.dot(x, ndw1_ref[h:], preferred_element_type=_F32))
    m = jnp.mean(zn, axis=0, keepdims=True)
    v = jnp.mean(zn * zn, axis=0, keepdims=True) - m * m
    hn = jnp.maximum((zn - m) * lax.rsqrt(v + 1e-5) * ndg_ref[...]
                     + ndbe_ref[...], 0.0)
    logit = jnp.dot(hn, ndw2_ref[...], preferred_element_type=_F32) \
        + ndb2_ref[...]
    nout_ref[...] = 1.0 / (1.0 + jnp.exp(-logit))
    c_ref[...] = (jnp.dot(agg, edw1_ref[:h], preferred_element_type=_F32)
                  + jnp.dot(x, edw1_ref[h:], preferred_element_type=_F32))


def _k5(sagg, x, ndw1, ndg, ndbe, ndw2, ndb2, edw1):
    n = x.shape[0]
    mid = ndw1.shape[1]
    return pl.pallas_call(
        _k5_body,
        out_shape=[jax.ShapeDtypeStruct((n, 1), _F32),
                   jax.ShapeDtypeStruct((n, mid), _F32)],
    )(sagg, x, ndw1, ndg, ndbe, ndw2, ndb2, edw1)


def _k7_body(ze_ref, scsh_ref, w2_ref, b2_ref, out_ref):
    he = jnp.maximum(ze_ref[...] * scsh_ref[0:1, :] + scsh_ref[1:2, :], 0.0)
    logit = jnp.dot(he, w2_ref[...], preferred_element_type=_F32) \
        + b2_ref[...]
    out_ref[...] = 1.0 / (1.0 + jnp.exp(-logit))


def _k7(ze, scsh, w2, b2, be=4000):
    e, h = ze.shape
    grid = e // be
    return pl.pallas_call(
        _k7_body,
        grid=(grid,),
        in_specs=[
            pl.BlockSpec((be, h), lambda i: (i, 0)),
            pl.BlockSpec((2, h), lambda i: (0, 0)),
            pl.BlockSpec((h, 1), lambda i: (0, 0)),
            pl.BlockSpec((1, 1), lambda i: (0, 0)),
        ],
        out_specs=pl.BlockSpec((be, 1), lambda i: (i, 0)),
        out_shape=jax.ShapeDtypeStruct((e, 1), _F32),
    )(ze, scsh, w2, b2)


def _scale_shift(sums, sumsq, e, g, be):
    m = sums / e
    v = sumsq / e - m * m
    sc = g * lax.rsqrt(v + 1e-5)
    return sc, be - m * sc


def _blockdiag4(w):
    h = w.shape[0]
    z = jnp.zeros((h, h), _F32)
    return jnp.block([[w, z, z, z], [z, w, z, z], [z, z, w, z],
                      [z, z, z, w]])


def _fold4(row):
    return row.reshape(4, 64).sum(axis=0)


def kernel(x, edge_index, ec_W1, ec_b1, ec_g1, ec_be1, ec_W2, ec_b2, ec_g2,
           ec_be2, ec_W3, ec_b3, ec_g3, ec_be3, nd_W1, nd_b1, nd_g1, nd_be1,
           nd_W2, nd_b2, ed_W1, ed_b1, ed_g1, ed_be1, ed_W2, ed_b2):
    del ec_b1, ec_b2, ec_b3, nd_b1, ed_b1
    e = edge_index.shape[1]
    ef = jnp.float32(e)
    dst = edge_index[1]

    t = _k0(x, ec_W1)
    z1p, st1 = _k1(t, edge_index)

    st1 = st1.reshape(NW, 128).sum(axis=0)
    sc1, sh1 = _scale_shift(st1[:64], st1[64:], ef, ec_g1, ec_be1)
    scsh1p = jnp.stack([jnp.tile(sc1, 4), jnp.tile(sh1, 4)])
    z2p, st2 = _mlp_layer(z1p, _blockdiag4(ec_W2), scsh1p)

    sc2, sh2 = _scale_shift(_fold4(st2[0]), _fold4(st2[1]), ef, ec_g2,
                            ec_be2)
    scsh2p = jnp.stack([jnp.tile(sc2, 4), jnp.tile(sh2, 4)])
    z3p, st3 = _mlp_layer(z2p, _blockdiag4(ec_W3), scsh2p)

    sc3, sh3 = _scale_shift(_fold4(st3[0]), _fold4(st3[1]), ef, ec_g3,
                            ec_be3)
    scsh3p = jnp.concatenate([jnp.tile(sc3, 4), jnp.tile(sh3, 4)])
    sagg = _k4(z3p, dst, scsh3p)

    node_out, c = _k5(sagg, x, nd_W1, nd_g1.reshape(1, -1),
                      nd_be1.reshape(1, -1), nd_W2, nd_b2.reshape(1, 1),
                      ed_W1)

    ze, ste = _k6(c, edge_index)
    ste = ste.reshape(NW, 256).sum(axis=0)
    sce, she = _scale_shift(ste[:128], ste[128:], ef, ed_g1, ed_be1)
    edge_out = _k7(ze, jnp.stack([sce, she]), ed_W2, ed_b2.reshape(1, 1))
    return (node_out, edge_out)
